# Initial kernel scaffold; baseline (speedup 1.0000x reference)
import functools

import jax
import jax.numpy as jnp
from jax import lax
from jax.experimental import pallas as pl
from jax.experimental.pallas import tpu as pltpu
from jax.experimental.pallas import tpu_sc as plsc

F32 = jnp.float32

SC_CORES = 2
SC_SUBCORES = 16
NW = SC_CORES * SC_SUBCORES
CG = 128
CS = 128


def _ln(v, g, b):
    mu = jnp.mean(v, axis=-1, keepdims=True)
    var = jnp.mean((v - mu) ** 2, axis=-1, keepdims=True)
    return (v - mu) * lax.rsqrt(var + 1e-5) * g + b


def _dot(a, b):
    return jnp.dot(a, b, preferred_element_type=F32)


def _node_proj(xp, posp, WiT, bi2, WlinT, WsaT, WdaT):
    npad, d = xp.shape
    blk = 1280
    grid = npad // blk

    def body(x_ref, pos_ref, wi, bi, wlin, wsa, wda, src_ref, dst_ref):
        x1 = jnp.maximum(_dot(x_ref[...], wi[...]) + bi[...], 0.0)
        src_ref[:, 0:128] = _dot(x1, wlin[...])
        src_ref[:, 128:192] = _dot(x1, wsa[...])
        src_ref[:, 192:208] = pos_ref[...]
        dst_ref[:, 0:64] = _dot(x1, wda[...])
        dst_ref[:, 64:80] = pos_ref[...]

    full = lambda s: pl.BlockSpec(s, lambda i: (0, 0))
    return pl.pallas_call(
        body,
        grid=(grid,),
        in_specs=[
            pl.BlockSpec((blk, d), lambda i: (i, 0)),
            pl.BlockSpec((blk, 16), lambda i: (i, 0)),
            full((d, d)), full((1, d)), full((d, d)), full((d, 64)), full((d, 64)),
        ],
        out_specs=[
            pl.BlockSpec((blk, 208), lambda i: (i, 0)),
            pl.BlockSpec((blk, 80), lambda i: (i, 0)),
        ],
        out_shape=[
            jax.ShapeDtypeStruct((npad, 208), F32),
            jax.ShapeDtypeStruct((npad, 80), F32),
        ],
    )(xp, posp, WiT, bi2, WlinT, WsaT, WdaT)


def _edge_mlp(gsrc, gdst, Wp1T, bp1, gp, betap, Wp2T, bp2, Wa1T, ba1, ga,
              betaa, Wa2T, ba2):
    epad = gsrc.shape[0]
    blk = 1024
    grid = epad // blk

    def body(gs_ref, gd_ref, wp1, bp1r, gpr, bpr, wp2, bp2r, wa1, ba1r, gar,
             bar, wa2, ba2r, msg_ref, ex_ref):
        gs = gs_ref[...]
        h = gs[:, 0:128]
        a1s = gs[:, 128:192]
        ps = gs[:, 192:208]
        gd = gd_ref[...]
        a1d = gd[:, 0:64]
        pd = gd[:, 64:80]
        t = _dot(pd - ps, wp1[...]) + bp1r[...]
        t = jnp.maximum(_ln(t, gpr[...], bpr[...]), 0.0)
        delta = _dot(t, wp2[...]) + bp2r[...]
        a = a1d - a1s + _dot(delta, wa1[...]) + ba1r[...]
        a = jnp.maximum(_ln(a, gar[...], bar[...]), 0.0)
        alpha = _dot(a, wa2[...]) + ba2r[...]
        ex = jnp.exp(alpha)
        ex_ref[...] = ex
        msg_ref[...] = ex * (h + delta)

    full = lambda s: pl.BlockSpec(s, lambda i: (0, 0))
    return pl.pallas_call(
        body,
        grid=(grid,),
        in_specs=[
            pl.BlockSpec((blk, 208), lambda i: (i, 0)),
            pl.BlockSpec((blk, 80), lambda i: (i, 0)),
            full((16, 64)), full((1, 64)), full((1, 64)), full((1, 64)),
            full((64, 128)), full((1, 128)),
            full((128, 64)), full((1, 64)), full((1, 64)), full((1, 64)),
            full((64, 128)), full((1, 128)),
        ],
        out_specs=[
            pl.BlockSpec((blk, 128), lambda i: (i, 0)),
            pl.BlockSpec((blk, 128), lambda i: (i, 0)),
        ],
        out_shape=[
            jax.ShapeDtypeStruct((epad, 128), F32),
            jax.ShapeDtypeStruct((epad, 128), F32),
        ],
    )(gsrc, gdst, Wp1T, bp1, gp, betap, Wp2T, bp2, Wa1T, ba1, ga, betaa,
      Wa2T, ba2)


def _final(num, den, WoT, bo2):
    npad, d = num.shape
    blk = 1280
    grid = npad // blk

    def body(n_ref, d_ref, wo, bo, out_ref):
        r = n_ref[...] / (d_ref[...] + 1e-16)
        out_ref[...] = jnp.maximum(_dot(r, wo[...]) + bo[...], 0.0)

    full = lambda s: pl.BlockSpec(s, lambda i: (0, 0))
    return pl.pallas_call(
        body,
        grid=(grid,),
        in_specs=[
            pl.BlockSpec((blk, d), lambda i: (i, 0)),
            pl.BlockSpec((blk, d), lambda i: (i, 0)),
            full((d, d)), full((1, d)),
        ],
        out_specs=pl.BlockSpec((blk, d), lambda i: (i, 0)),
        out_shape=jax.ShapeDtypeStruct((npad, d), F32),
    )(num, den, WoT, bo2)


def _sc_gather(src_ids, dst_ids, src_tab, dst_tab):
    epad = src_ids.shape[0]
    ws = src_tab.shape[1]
    wd = dst_tab.shape[1]
    bpw = epad // NW
    nch = bpw // CG
    mesh = plsc.VectorSubcoreMesh(core_axis_name="c", subcore_axis_name="s")

    @functools.partial(
        pl.kernel,
        out_type=(
            jax.ShapeDtypeStruct((epad, ws), F32),
            jax.ShapeDtypeStruct((epad, wd), F32),
        ),
        mesh=mesh,
        scratch_types=[
            pltpu.VMEM((CG,), jnp.int32),
            pltpu.VMEM((CG,), jnp.int32),
            pltpu.VMEM((CG, ws), F32),
            pltpu.VMEM((CG, wd), F32),
            pltpu.SemaphoreType.DMA,
        ],
    )
    def k(sid_h, did_h, stab_h, dtab_h, gsrc_h, gdst_h, idx_s, idx_d, buf_s,
          buf_d, sem):
        wid = lax.axis_index("s") * SC_CORES + lax.axis_index("c")
        base = wid * bpw

        def body(i, carry):
            off = base + i * CG
            pltpu.sync_copy(sid_h.at[pl.ds(off, CG)], idx_s)
            pltpu.sync_copy(did_h.at[pl.ds(off, CG)], idx_d)
            d1 = pltpu.async_copy(stab_h.at[idx_s], buf_s, sem)
            d2 = pltpu.async_copy(dtab_h.at[idx_d], buf_d, sem)
            d1.wait()
            d2.wait()
            pltpu.sync_copy(buf_s, gsrc_h.at[pl.ds(off, CG)])
            pltpu.sync_copy(buf_d, gdst_h.at[pl.ds(off, CG)])
            return carry

        lax.fori_loop(0, nch, body, 0)

    return k(src_ids, dst_ids, src_tab, dst_tab)


def _sc_scatter(msg, ex, dst_ids, npad):
    epad, d = msg.shape
    per_tile = epad // SC_SUBCORES
    nch = per_tile // CS
    rows_per_tile = npad // SC_SUBCORES
    mesh = plsc.VectorSubcoreMesh(core_axis_name="c", subcore_axis_name="s")

    @functools.partial(
        pl.kernel,
        out_type=(
            jax.ShapeDtypeStruct((npad, d), F32),
            jax.ShapeDtypeStruct((npad, d), F32),
        ),
        mesh=mesh,
        scratch_types=[
            pltpu.VMEM((CS, d), F32),
            pltpu.VMEM((CS,), jnp.int32),
            pltpu.VMEM((16, d), F32),
            pltpu.VMEM_SHARED((npad, d), F32),
        ],
    )
    def k(msg_h, ex_h, did_h, num_h, den_h, rowbuf, idxb, zbuf, acc):
        c = lax.axis_index("c")
        s = lax.axis_index("s")

        for j in range(16):
            for t in range(d // 16):
                zbuf[j, pl.ds(t * 16, 16)] = jnp.zeros((16,), F32)

        def zero_body(i, carry):
            pltpu.sync_copy(zbuf, acc.at[pl.ds(s * rows_per_tile + i * 16, 16)])
            return carry

        lax.fori_loop(0, rows_per_tile // 16, zero_body, 0)
        plsc.subcore_barrier()

        def process(tbl_h):
            def body(i, carry):
                off = s * per_tile + i * CS
                pltpu.sync_copy(tbl_h.at[pl.ds(off, CS)], rowbuf)
                pltpu.sync_copy(did_h.at[pl.ds(off, CS)], idxb)
                pltpu.sync_copy(rowbuf, acc.at[idxb], add=True)
                return carry

            lax.fori_loop(0, nch, body, 0)

        pl.when(c == 0)(lambda: process(msg_h))
        pl.when(c == 1)(lambda: process(ex_h))
        plsc.subcore_barrier()

        def dump(out_h):
            pltpu.sync_copy(
                acc.at[pl.ds(s * rows_per_tile, rows_per_tile)],
                out_h.at[pl.ds(s * rows_per_tile, rows_per_tile)])

        pl.when(c == 0)(lambda: dump(num_h))
        pl.when(c == 1)(lambda: dump(den_h))

    return k(msg, ex, dst_ids)


def kernel(x, pos, edge_index, Wi, bi, Wo, bo, Wlin, Wsrc, Wdst, Wp1, bp1,
           gp, betap, Wp2, bp2, Wa1, ba1, ga, betaa, Wa2, ba2):
    n, dm = x.shape
    e = edge_index.shape[1]
    etot = e + n

    npad = ((n + 1 + 255) // 256) * 256
    unit = NW * CG
    epad = ((etot + unit - 1) // unit) * unit

    loop = jnp.arange(n, dtype=jnp.int32)
    pad_e = epad - etot
    src_full = jnp.concatenate([
        edge_index[0].astype(jnp.int32), loop,
        jnp.zeros((pad_e,), jnp.int32)])
    dst_full = jnp.concatenate([
        edge_index[1].astype(jnp.int32), loop,
        n + (jnp.arange(pad_e, dtype=jnp.int32) % (npad - n))])

    xp = jnp.pad(x, ((0, npad - n), (0, 0)))
    posp = jnp.pad(pos, ((0, npad - n), (0, 16 - pos.shape[1])))
    WiT = Wi.T
    WlinT = Wlin.T
    WsaT = (Wa1 @ Wsrc).T
    WdaT = (Wa1 @ Wdst).T
    Wp1T = jnp.pad(Wp1.T, ((0, 16 - Wp1.shape[1]), (0, 0)))
    Wp2T = Wp2.T
    Wa1T = Wa1.T
    Wa2T = Wa2.T
    WoT = Wo.T
    r1 = lambda v: v.reshape(1, -1)

    srctab, dsttab = _node_proj(xp, posp, WiT, r1(bi), WlinT, WsaT, WdaT)
    gsrc, gdst = _sc_gather(src_full, dst_full, srctab, dsttab)
    msg, ex = _edge_mlp(gsrc, gdst, Wp1T, r1(bp1), r1(gp), r1(betap), Wp2T,
                        r1(bp2), Wa1T, r1(ba1), r1(ga), r1(betaa), Wa2T,
                        r1(ba2))
    num, den = _sc_scatter(msg, ex, dst_full, npad)
    out = _final(num, den, WoT, r1(bo))
    return out[:n]


# speedup vs baseline: 6.4163x; 6.4163x over previous
import functools

import jax
import jax.numpy as jnp
from jax import lax
from jax.experimental import pallas as pl
from jax.experimental.pallas import tpu as pltpu
from jax.experimental.pallas import tpu_sc as plsc

F32 = jnp.float32

SC_CORES = 2
SC_SUBCORES = 16
NW = SC_CORES * SC_SUBCORES
CG = 128
CS = 128


def _ln(v, g, b):
    mu = jnp.mean(v, axis=-1, keepdims=True)
    var = jnp.mean((v - mu) ** 2, axis=-1, keepdims=True)
    return (v - mu) * lax.rsqrt(var + 1e-5) * g + b


def _dot(a, b):
    return jnp.dot(a, b, preferred_element_type=F32)


def _node_proj(xp, posp, WiT, bi2, WlinT, WsaT, WdaT):
    npad, d = xp.shape
    blk = 1280
    grid = npad // blk

    def body(x_ref, pos_ref, wi, bi, wlin, wsa, wda, src_ref, dst_ref):
        x1 = jnp.maximum(_dot(x_ref[...], wi[...]) + bi[...], 0.0)
        src_ref[:, 0:128] = _dot(x1, wlin[...])
        src_ref[:, 128:192] = _dot(x1, wsa[...])
        src_ref[:, 192:208] = pos_ref[...]
        src_ref[:, 208:256] = jnp.zeros((blk, 48), F32)
        dst_ref[:, 0:64] = _dot(x1, wda[...])
        dst_ref[:, 64:80] = pos_ref[...]
        dst_ref[:, 80:128] = jnp.zeros((blk, 48), F32)

    full = lambda s: pl.BlockSpec(s, lambda i: (0, 0))
    return pl.pallas_call(
        body,
        grid=(grid,),
        in_specs=[
            pl.BlockSpec((blk, d), lambda i: (i, 0)),
            pl.BlockSpec((blk, 16), lambda i: (i, 0)),
            full((d, d)), full((1, d)), full((d, d)), full((d, 64)), full((d, 64)),
        ],
        out_specs=[
            pl.BlockSpec((blk, 256), lambda i: (i, 0)),
            pl.BlockSpec((blk, 128), lambda i: (i, 0)),
        ],
        out_shape=[
            jax.ShapeDtypeStruct((npad, 256), F32),
            jax.ShapeDtypeStruct((npad, 128), F32),
        ],
    )(xp, posp, WiT, bi2, WlinT, WsaT, WdaT)


def _edge_mlp(gsrc, gdst, Wp1T, bp1, gp, betap, Wp2T, bp2, Wa1T, ba1, ga,
              betaa, Wa2T, ba2):
    epad = gsrc.shape[0]
    blk = 1024
    grid = epad // blk

    def body(gs_ref, gd_ref, wp1, bp1r, gpr, bpr, wp2, bp2r, wa1, ba1r, gar,
             bar, wa2, ba2r, msg_ref, ex_ref):
        gs = gs_ref[...]
        h = gs[:, 0:128]
        a1s = gs[:, 128:192]
        ps = gs[:, 192:208]
        gd = gd_ref[...]
        a1d = gd[:, 0:64]
        pd = gd[:, 64:80]
        t = _dot(pd - ps, wp1[...]) + bp1r[...]
        t = jnp.maximum(_ln(t, gpr[...], bpr[...]), 0.0)
        delta = _dot(t, wp2[...]) + bp2r[...]
        a = a1d - a1s + _dot(delta, wa1[...]) + ba1r[...]
        a = jnp.maximum(_ln(a, gar[...], bar[...]), 0.0)
        alpha = _dot(a, wa2[...]) + ba2r[...]
        ex = jnp.exp(alpha)
        ex_ref[...] = ex
        msg_ref[...] = ex * (h + delta)

    full = lambda s: pl.BlockSpec(s, lambda i: (0, 0))
    return pl.pallas_call(
        body,
        grid=(grid,),
        in_specs=[
            pl.BlockSpec((blk, 256), lambda i: (i, 0)),
            pl.BlockSpec((blk, 128), lambda i: (i, 0)),
            full((16, 64)), full((1, 64)), full((1, 64)), full((1, 64)),
            full((64, 128)), full((1, 128)),
            full((128, 64)), full((1, 64)), full((1, 64)), full((1, 64)),
            full((64, 128)), full((1, 128)),
        ],
        out_specs=[
            pl.BlockSpec((blk, 128), lambda i: (i, 0)),
            pl.BlockSpec((blk, 128), lambda i: (i, 0)),
        ],
        out_shape=[
            jax.ShapeDtypeStruct((epad, 128), F32),
            jax.ShapeDtypeStruct((epad, 128), F32),
        ],
    )(gsrc, gdst, Wp1T, bp1, gp, betap, Wp2T, bp2, Wa1T, ba1, ga, betaa,
      Wa2T, ba2)


def _final(num, den, WoT, bo2):
    npad, d = num.shape
    blk = 1280
    grid = npad // blk

    def body(n_ref, d_ref, wo, bo, out_ref):
        r = n_ref[...] / (d_ref[...] + 1e-16)
        out_ref[...] = jnp.maximum(_dot(r, wo[...]) + bo[...], 0.0)

    full = lambda s: pl.BlockSpec(s, lambda i: (0, 0))
    return pl.pallas_call(
        body,
        grid=(grid,),
        in_specs=[
            pl.BlockSpec((blk, d), lambda i: (i, 0)),
            pl.BlockSpec((blk, d), lambda i: (i, 0)),
            full((d, d)), full((1, d)),
        ],
        out_specs=pl.BlockSpec((blk, d), lambda i: (i, 0)),
        out_shape=jax.ShapeDtypeStruct((npad, d), F32),
    )(num, den, WoT, bo2)


def _sc_gather(src_ids, dst_ids, src_tab, dst_tab):
    epad = src_ids.shape[0]
    ws = src_tab.shape[1]
    wd = dst_tab.shape[1]
    bpw = epad // NW
    nch = bpw // CG
    mesh = plsc.VectorSubcoreMesh(core_axis_name="c", subcore_axis_name="s")

    @functools.partial(
        pl.kernel,
        out_type=(
            jax.ShapeDtypeStruct((epad, ws), F32),
            jax.ShapeDtypeStruct((epad, wd), F32),
        ),
        mesh=mesh,
        scratch_types=[
            pltpu.VMEM((CG,), jnp.int32),
            pltpu.VMEM((CG,), jnp.int32),
            pltpu.VMEM((CG, ws), F32),
            pltpu.VMEM((CG, wd), F32),
            pltpu.SemaphoreType.DMA,
        ],
    )
    def k(sid_h, did_h, stab_h, dtab_h, gsrc_h, gdst_h, idx_s, idx_d, buf_s,
          buf_d, sem):
        wid = lax.axis_index("s") * SC_CORES + lax.axis_index("c")
        base = wid * bpw

        def body(i, carry):
            off = base + i * CG
            pltpu.sync_copy(sid_h.at[pl.ds(off, CG)], idx_s)
            pltpu.sync_copy(did_h.at[pl.ds(off, CG)], idx_d)
            d1 = pltpu.async_copy(stab_h.at[idx_s], buf_s, sem)
            d2 = pltpu.async_copy(dtab_h.at[idx_d], buf_d, sem)
            d1.wait()
            d2.wait()
            pltpu.sync_copy(buf_s, gsrc_h.at[pl.ds(off, CG)])
            pltpu.sync_copy(buf_d, gdst_h.at[pl.ds(off, CG)])
            return carry

        lax.fori_loop(0, nch, body, 0)

    return k(src_ids, dst_ids, src_tab, dst_tab)


def _sc_scatter(msg, ex, dst_ids, npad):
    epad, d = msg.shape
    per_tile = epad // SC_SUBCORES
    nch = per_tile // CS
    rows_per_tile = npad // SC_SUBCORES
    mesh = plsc.VectorSubcoreMesh(core_axis_name="c", subcore_axis_name="s")

    @functools.partial(
        pl.kernel,
        out_type=(
            jax.ShapeDtypeStruct((npad, d), F32),
            jax.ShapeDtypeStruct((npad, d), F32),
        ),
        mesh=mesh,
        scratch_types=[
            pltpu.VMEM((CS, d), F32),
            pltpu.VMEM((CS,), jnp.int32),
            pltpu.VMEM((16, d), F32),
            pltpu.VMEM_SHARED((npad, d), F32),
        ],
    )
    def k(msg_h, ex_h, did_h, num_h, den_h, rowbuf, idxb, zbuf, acc):
        c = lax.axis_index("c")
        s = lax.axis_index("s")

        for j in range(16):
            for t in range(d // 16):
                zbuf[j, pl.ds(t * 16, 16)] = jnp.zeros((16,), F32)

        def zero_body(i, carry):
            pltpu.sync_copy(zbuf, acc.at[pl.ds(s * rows_per_tile + i * 16, 16)])
            return carry

        lax.fori_loop(0, rows_per_tile // 16, zero_body, 0)
        plsc.subcore_barrier()

        def process(tbl_h):
            def body(i, carry):
                off = s * per_tile + i * CS
                pltpu.sync_copy(tbl_h.at[pl.ds(off, CS)], rowbuf)
                pltpu.sync_copy(did_h.at[pl.ds(off, CS)], idxb)
                pltpu.sync_copy(rowbuf, acc.at[idxb], add=True)
                return carry

            lax.fori_loop(0, nch, body, 0)

        pl.when(c == 0)(lambda: process(msg_h))
        pl.when(c == 1)(lambda: process(ex_h))
        plsc.subcore_barrier()

        def dump(out_h):
            pltpu.sync_copy(
                acc.at[pl.ds(s * rows_per_tile, rows_per_tile)],
                out_h.at[pl.ds(s * rows_per_tile, rows_per_tile)])

        pl.when(c == 0)(lambda: dump(num_h))
        pl.when(c == 1)(lambda: dump(den_h))

    return k(msg, ex, dst_ids)


def kernel(x, pos, edge_index, Wi, bi, Wo, bo, Wlin, Wsrc, Wdst, Wp1, bp1,
           gp, betap, Wp2, bp2, Wa1, ba1, ga, betaa, Wa2, ba2):
    n, dm = x.shape
    e = edge_index.shape[1]
    etot = e + n

    npad = ((n + 1 + 255) // 256) * 256
    unit = NW * CG
    epad = ((etot + unit - 1) // unit) * unit

    loop = jnp.arange(n, dtype=jnp.int32)
    pad_e = epad - etot
    src_full = jnp.concatenate([
        edge_index[0].astype(jnp.int32), loop,
        jnp.zeros((pad_e,), jnp.int32)])
    dst_full = jnp.concatenate([
        edge_index[1].astype(jnp.int32), loop,
        n + (jnp.arange(pad_e, dtype=jnp.int32) % (npad - n))])

    xp = jnp.pad(x, ((0, npad - n), (0, 0)))
    posp = jnp.pad(pos, ((0, npad - n), (0, 16 - pos.shape[1])))
    WiT = Wi.T
    WlinT = Wlin.T
    WsaT = (Wa1 @ Wsrc).T
    WdaT = (Wa1 @ Wdst).T
    Wp1T = jnp.pad(Wp1.T, ((0, 16 - Wp1.shape[1]), (0, 0)))
    Wp2T = Wp2.T
    Wa1T = Wa1.T
    Wa2T = Wa2.T
    WoT = Wo.T
    r1 = lambda v: v.reshape(1, -1)

    srctab, dsttab = _node_proj(xp, posp, WiT, r1(bi), WlinT, WsaT, WdaT)
    gsrc, gdst = _sc_gather(src_full, dst_full, srctab, dsttab)
    msg, ex = _edge_mlp(gsrc, gdst, Wp1T, r1(bp1), r1(gp), r1(betap), Wp2T,
                        r1(bp2), Wa1T, r1(ba1), r1(ga), r1(betaa), Wa2T,
                        r1(ba2))
    num, den = _sc_scatter(msg, ex, dst_full, npad)
    out = _final(num, den, WoT, r1(bo))
    return out[:n]


# speedup vs baseline: 7.4905x; 1.1674x over previous
import functools

import jax
import jax.numpy as jnp
from jax import lax
from jax.experimental import pallas as pl
from jax.experimental.pallas import tpu as pltpu
from jax.experimental.pallas import tpu_sc as plsc

F32 = jnp.float32

SC_CORES = 2
SC_SUBCORES = 16
NW = SC_CORES * SC_SUBCORES
CG = 128
CS = 128


def _ln(v, g, b):
    mu = jnp.mean(v, axis=-1, keepdims=True)
    var = jnp.mean((v - mu) ** 2, axis=-1, keepdims=True)
    return (v - mu) * lax.rsqrt(var + 1e-5) * g + b


def _dot(a, b):
    return jnp.dot(a, b, preferred_element_type=F32)


def _node_proj(xp, posp, WiT, bi2, WlinT, WsaT, WdaT):
    npad, d = xp.shape
    blk = 1280
    grid = npad // blk

    def body(x_ref, pos_ref, wi, bi, wlin, wsa, wda, src_ref, dst_ref):
        x1 = jnp.maximum(_dot(x_ref[...], wi[...]) + bi[...], 0.0)
        src_ref[:, 0:128] = _dot(x1, wlin[...])
        src_ref[:, 128:192] = _dot(x1, wsa[...])
        src_ref[:, 192:208] = pos_ref[...]
        src_ref[:, 208:256] = jnp.zeros((blk, 48), F32)
        dst_ref[:, 0:64] = _dot(x1, wda[...])
        dst_ref[:, 64:80] = pos_ref[...]
        dst_ref[:, 80:128] = jnp.zeros((blk, 48), F32)

    full = lambda s: pl.BlockSpec(s, lambda i: (0, 0))
    return pl.pallas_call(
        body,
        grid=(grid,),
        in_specs=[
            pl.BlockSpec((blk, d), lambda i: (i, 0)),
            pl.BlockSpec((blk, 16), lambda i: (i, 0)),
            full((d, d)), full((1, d)), full((d, d)), full((d, 64)), full((d, 64)),
        ],
        out_specs=[
            pl.BlockSpec((blk, 256), lambda i: (i, 0)),
            pl.BlockSpec((blk, 128), lambda i: (i, 0)),
        ],
        out_shape=[
            jax.ShapeDtypeStruct((npad, 256), F32),
            jax.ShapeDtypeStruct((npad, 128), F32),
        ],
    )(xp, posp, WiT, bi2, WlinT, WsaT, WdaT)


def _edge_mlp(gsrc, gdst, Wp1T, bp1, gp, betap, Wp2T, bp2, Wa1T, ba1, ga,
              betaa, Wa2T, ba2):
    epad = gsrc.shape[0]
    blk = 1024
    grid = epad // blk

    def body(gs_ref, gd_ref, wp1, bp1r, gpr, bpr, wp2, bp2r, wa1, ba1r, gar,
             bar, wa2, ba2r, msg_ref, ex_ref):
        gs = gs_ref[...]
        h = gs[:, 0:128]
        a1s = gs[:, 128:192]
        ps = gs[:, 192:208]
        gd = gd_ref[...]
        a1d = gd[:, 0:64]
        pd = gd[:, 64:80]
        t = _dot(pd - ps, wp1[...]) + bp1r[...]
        t = jnp.maximum(_ln(t, gpr[...], bpr[...]), 0.0)
        delta = _dot(t, wp2[...]) + bp2r[...]
        a = a1d - a1s + _dot(delta, wa1[...]) + ba1r[...]
        a = jnp.maximum(_ln(a, gar[...], bar[...]), 0.0)
        alpha = _dot(a, wa2[...]) + ba2r[...]
        ex = jnp.exp(alpha)
        ex_ref[...] = ex
        msg_ref[...] = ex * (h + delta)

    full = lambda s: pl.BlockSpec(s, lambda i: (0, 0))
    return pl.pallas_call(
        body,
        grid=(grid,),
        in_specs=[
            pl.BlockSpec((blk, 256), lambda i: (i, 0)),
            pl.BlockSpec((blk, 128), lambda i: (i, 0)),
            full((16, 64)), full((1, 64)), full((1, 64)), full((1, 64)),
            full((64, 128)), full((1, 128)),
            full((128, 64)), full((1, 64)), full((1, 64)), full((1, 64)),
            full((64, 128)), full((1, 128)),
        ],
        out_specs=[
            pl.BlockSpec((blk, 128), lambda i: (i, 0)),
            pl.BlockSpec((blk, 128), lambda i: (i, 0)),
        ],
        out_shape=[
            jax.ShapeDtypeStruct((epad, 128), F32),
            jax.ShapeDtypeStruct((epad, 128), F32),
        ],
    )(gsrc, gdst, Wp1T, bp1, gp, betap, Wp2T, bp2, Wa1T, ba1, ga, betaa,
      Wa2T, ba2)


def _final(num, den, WoT, bo2):
    npad, d = num.shape
    blk = 1280
    grid = npad // blk

    def body(n_ref, d_ref, wo, bo, out_ref):
        r = n_ref[...] / (d_ref[...] + 1e-16)
        out_ref[...] = jnp.maximum(_dot(r, wo[...]) + bo[...], 0.0)

    full = lambda s: pl.BlockSpec(s, lambda i: (0, 0))
    return pl.pallas_call(
        body,
        grid=(grid,),
        in_specs=[
            pl.BlockSpec((blk, d), lambda i: (i, 0)),
            pl.BlockSpec((blk, d), lambda i: (i, 0)),
            full((d, d)), full((1, d)),
        ],
        out_specs=pl.BlockSpec((blk, d), lambda i: (i, 0)),
        out_shape=jax.ShapeDtypeStruct((npad, d), F32),
    )(num, den, WoT, bo2)


def _sc_gather(src_ids, dst_ids, src_tab, dst_tab):
    epad = src_ids.shape[0]
    ws = src_tab.shape[1]
    wd = dst_tab.shape[1]
    bpw = epad // NW
    nch = bpw // CG
    mesh = plsc.VectorSubcoreMesh(core_axis_name="c", subcore_axis_name="s")

    assert nch % 2 == 1 and nch >= 3, nch
    npairs = (nch - 1) // 2

    @functools.partial(
        pl.kernel,
        out_type=(
            jax.ShapeDtypeStruct((epad, ws), F32),
            jax.ShapeDtypeStruct((epad, wd), F32),
        ),
        mesh=mesh,
        scratch_types=[
            [pltpu.VMEM((CG,), jnp.int32)] * 2,
            [pltpu.VMEM((CG,), jnp.int32)] * 2,
            [pltpu.VMEM((CG, ws), F32)] * 2,
            [pltpu.VMEM((CG, wd), F32)] * 2,
            [pltpu.SemaphoreType.DMA] * 2,
            [pltpu.SemaphoreType.DMA] * 2,
        ],
    )
    def k(sid_h, did_h, stab_h, dtab_h, gsrc_h, gdst_h, idx_s, idx_d, buf_s,
          buf_d, sem_g, sem_w):
        wid = lax.axis_index("s") * SC_CORES + lax.axis_index("c")
        base = wid * bpw

        def g_start(off, t):
            pltpu.sync_copy(sid_h.at[pl.ds(off, CG)], idx_s[t])
            pltpu.sync_copy(did_h.at[pl.ds(off, CG)], idx_d[t])
            pltpu.async_copy(stab_h.at[idx_s[t]], buf_s[t], sem_g[t])
            pltpu.async_copy(dtab_h.at[idx_d[t]], buf_d[t], sem_g[t])

        def g_wait(t):
            pltpu.make_async_copy(stab_h.at[idx_s[t]], buf_s[t], sem_g[t]).wait()
            pltpu.make_async_copy(dtab_h.at[idx_d[t]], buf_d[t], sem_g[t]).wait()

        def w_start(off, t):
            pltpu.async_copy(buf_s[t], gsrc_h.at[pl.ds(off, CG)], sem_w[t])
            pltpu.async_copy(buf_d[t], gdst_h.at[pl.ds(off, CG)], sem_w[t])

        def w_wait(off, t):
            pltpu.make_async_copy(buf_s[t], gsrc_h.at[pl.ds(off, CG)], sem_w[t]).wait()
            pltpu.make_async_copy(buf_d[t], gdst_h.at[pl.ds(off, CG)], sem_w[t]).wait()

        g_start(base, 0)

        def body(p, carry):
            offa = base + (2 * p) * CG
            offb = offa + CG
            offn = offb + CG
            g_wait(0)
            pl.when(p > 0)(lambda: w_wait(offa - CG, 1))
            g_start(offb, 1)
            w_start(offa, 0)
            g_wait(1)
            w_start(offb, 1)
            w_wait(offa, 0)
            g_start(offn, 0)
            return carry

        lax.fori_loop(0, npairs, body, 0)
        last = base + (nch - 1) * CG
        g_wait(0)
        w_wait(last - CG, 1)
        w_start(last, 0)
        w_wait(last, 0)

    return k(src_ids, dst_ids, src_tab, dst_tab)


def _sc_scatter(msg, ex, dst_ids, npad):
    epad, d = msg.shape
    per_tile = epad // SC_SUBCORES
    nch = per_tile // CS
    rows_per_tile = npad // SC_SUBCORES
    mesh = plsc.VectorSubcoreMesh(core_axis_name="c", subcore_axis_name="s")

    @functools.partial(
        pl.kernel,
        out_type=(
            jax.ShapeDtypeStruct((npad, d), F32),
            jax.ShapeDtypeStruct((npad, d), F32),
        ),
        mesh=mesh,
        scratch_types=[
            [pltpu.VMEM((CS, d), F32)] * 2,
            [pltpu.VMEM((CS,), jnp.int32)] * 2,
            pltpu.VMEM((64, d), F32),
            pltpu.VMEM_SHARED((npad, d), F32),
            [pltpu.SemaphoreType.DMA] * 2,
            [pltpu.SemaphoreType.DMA] * 2,
        ],
    )
    def k(msg_h, ex_h, did_h, num_h, den_h, rowbuf, idxb, zbuf, acc, sem_l,
          sem_a):
        c = lax.axis_index("c")
        s = lax.axis_index("s")

        for j in range(64):
            for t in range(d // 16):
                zbuf[j, pl.ds(t * 16, 16)] = jnp.zeros((16,), F32)

        def zero_body(i, carry):
            pltpu.sync_copy(zbuf, acc.at[pl.ds(s * rows_per_tile + i * 64, 64)])
            return carry

        lax.fori_loop(0, rows_per_tile // 64, zero_body, 0)
        plsc.subcore_barrier()

        def process(tbl_h):
            base = s * per_tile

            def l_start(off, t):
                pltpu.sync_copy(did_h.at[pl.ds(off, CS)], idxb[t])
                pltpu.async_copy(tbl_h.at[pl.ds(off, CS)], rowbuf[t], sem_l[t])

            def l_wait(off, t):
                pltpu.make_async_copy(
                    tbl_h.at[pl.ds(off, CS)], rowbuf[t], sem_l[t]).wait()

            def a_start(t):
                pltpu.async_copy(rowbuf[t], acc.at[idxb[t]], sem_a[t], add=True)

            def a_wait(t):
                pltpu.make_async_copy(
                    rowbuf[t], acc.at[idxb[t]], sem_a[t]).wait()

            l_start(base, 0)

            def body(p, carry):
                offa = base + (2 * p) * CS
                offb = offa + CS
                offn = offb + CS
                pl.when(p > 0)(lambda: a_wait(1))
                l_start(offb, 1)
                l_wait(offa, 0)
                a_start(0)
                l_wait(offb, 1)
                a_wait(0)
                pl.when(p + 1 < nch // 2)(lambda: l_start(offn, 0))
                a_start(1)
                return carry

            lax.fori_loop(0, nch // 2, body, 0)
            a_wait(1)

        pl.when(c == 0)(lambda: process(msg_h))
        pl.when(c == 1)(lambda: process(ex_h))
        plsc.subcore_barrier()

        def dump(out_h):
            pltpu.sync_copy(
                acc.at[pl.ds(s * rows_per_tile, rows_per_tile)],
                out_h.at[pl.ds(s * rows_per_tile, rows_per_tile)])

        pl.when(c == 0)(lambda: dump(num_h))
        pl.when(c == 1)(lambda: dump(den_h))

    return k(msg, ex, dst_ids)


def kernel(x, pos, edge_index, Wi, bi, Wo, bo, Wlin, Wsrc, Wdst, Wp1, bp1,
           gp, betap, Wp2, bp2, Wa1, ba1, ga, betaa, Wa2, ba2):
    n, dm = x.shape
    e = edge_index.shape[1]
    etot = e + n

    npad = ((n + 1 + 255) // 256) * 256
    unit = NW * CG
    epad = ((etot + unit - 1) // unit) * unit

    loop = jnp.arange(n, dtype=jnp.int32)
    pad_e = epad - etot
    src_full = jnp.concatenate([
        edge_index[0].astype(jnp.int32), loop,
        jnp.zeros((pad_e,), jnp.int32)])
    dst_full = jnp.concatenate([
        edge_index[1].astype(jnp.int32), loop,
        n + (jnp.arange(pad_e, dtype=jnp.int32) % (npad - n))])

    xp = jnp.pad(x, ((0, npad - n), (0, 0)))
    posp = jnp.pad(pos, ((0, npad - n), (0, 16 - pos.shape[1])))
    WiT = Wi.T
    WlinT = Wlin.T
    WsaT = (Wa1 @ Wsrc).T
    WdaT = (Wa1 @ Wdst).T
    Wp1T = jnp.pad(Wp1.T, ((0, 16 - Wp1.shape[1]), (0, 0)))
    Wp2T = Wp2.T
    Wa1T = Wa1.T
    Wa2T = Wa2.T
    WoT = Wo.T
    r1 = lambda v: v.reshape(1, -1)

    srctab, dsttab = _node_proj(xp, posp, WiT, r1(bi), WlinT, WsaT, WdaT)
    gsrc, gdst = _sc_gather(src_full, dst_full, srctab, dsttab)
    msg, ex = _edge_mlp(gsrc, gdst, Wp1T, r1(bp1), r1(gp), r1(betap), Wp2T,
                        r1(bp2), Wa1T, r1(ba1), r1(ga), r1(betaa), Wa2T,
                        r1(ba2))
    num, den = _sc_scatter(msg, ex, dst_full, npad)
    out = _final(num, den, WoT, r1(bo))
    return out[:n]


# speedup vs baseline: 7.8960x; 1.0541x over previous
import functools

import jax
import jax.numpy as jnp
from jax import lax
from jax.experimental import pallas as pl
from jax.experimental.pallas import tpu as pltpu
from jax.experimental.pallas import tpu_sc as plsc

F32 = jnp.float32
BF16 = jnp.bfloat16

SC_CORES = 2
SC_SUBCORES = 16
NW = SC_CORES * SC_SUBCORES
CG = 128
CS = 128


def _ln(v, g, b):
    mu = jnp.mean(v, axis=-1, keepdims=True)
    var = jnp.mean((v - mu) ** 2, axis=-1, keepdims=True)
    return (v - mu) * lax.rsqrt(var + 1e-5) * g + b


def _dot(a, b):
    return jnp.dot(a, b, preferred_element_type=F32)


def _pack2(a, b):
    ua = (lax.bitcast_convert_type(a, jnp.uint32) + jnp.uint32(0x8000)) >> 16
    ub = (lax.bitcast_convert_type(b, jnp.uint32) + jnp.uint32(0x8000)) & jnp.uint32(0xFFFF0000)
    return lax.bitcast_convert_type(ua | ub, jnp.int32)


def _unpack2(u):
    uu = lax.bitcast_convert_type(u, jnp.uint32)
    a = lax.bitcast_convert_type(uu << 16, F32)
    b = lax.bitcast_convert_type(uu & jnp.uint32(0xFFFF0000), F32)
    return a, b


def _node_proj(xp, posp, WiT, bi2, WlinT, WsaT, WdaT):
    npad, d = xp.shape
    blk = 1280
    grid = npad // blk

    def body(x_ref, pos_ref, wi, bi, wlin, wsa, wda, src_ref, dst_ref):
        x1 = jnp.maximum(_dot(x_ref[...], wi[...]) + bi[...], 0.0)
        h = _dot(x1, wlin[...])
        a1s = _dot(x1, wsa[...])
        psv = pos_ref[...]
        src_ref[:, 0:64] = _pack2(h[:, 0:64], h[:, 64:128])
        src_ref[:, 64:96] = _pack2(a1s[:, 0:32], a1s[:, 32:64])
        src_ref[:, 96:104] = _pack2(psv[:, 0:8], psv[:, 8:16])
        src_ref[:, 104:128] = jnp.zeros((blk, 24), jnp.int32)
        dst_ref[:, 0:64] = _dot(x1, wda[...])
        dst_ref[:, 64:80] = pos_ref[...]
        dst_ref[:, 80:128] = jnp.zeros((blk, 48), F32)

    full = lambda s: pl.BlockSpec(s, lambda i: (0, 0))
    return pl.pallas_call(
        body,
        grid=(grid,),
        in_specs=[
            pl.BlockSpec((blk, d), lambda i: (i, 0)),
            pl.BlockSpec((blk, 16), lambda i: (i, 0)),
            full((d, d)), full((1, d)), full((d, d)), full((d, 64)), full((d, 64)),
        ],
        out_specs=[
            pl.BlockSpec((blk, 128), lambda i: (i, 0)),
            pl.BlockSpec((blk, 128), lambda i: (i, 0)),
        ],
        out_shape=[
            jax.ShapeDtypeStruct((npad, 128), jnp.int32),
            jax.ShapeDtypeStruct((npad, 128), F32),
        ],
    )(xp, posp, WiT, bi2, WlinT, WsaT, WdaT)


def _edge_mlp(gsrc, gdst, Wp1T, bp1, gp, betap, Wp2T, bp2, Wa1T, ba1, ga,
              betaa, Wa2T, ba2):
    epad = gsrc.shape[0]
    blk = 1024
    grid = epad // blk

    def body(gs_ref, gd_ref, wp1, bp1r, gpr, bpr, wp2, bp2r, wa1, ba1r, gar,
             bar, wa2, ba2r, msg_ref, ex_ref):
        u = gs_ref[...]
        h0, h1 = _unpack2(u[:, 0:64])
        h = jnp.concatenate([h0, h1], axis=-1)
        s0, s1 = _unpack2(u[:, 64:96])
        a1s = jnp.concatenate([s0, s1], axis=-1)
        p0, p1 = _unpack2(u[:, 96:104])
        ps = jnp.concatenate([p0, p1], axis=-1)
        gd = gd_ref[...]
        a1d = gd[:, 0:64]
        pd = gd[:, 64:80]
        t = _dot(pd - ps, wp1[...]) + bp1r[...]
        t = jnp.maximum(_ln(t, gpr[...], bpr[...]), 0.0)
        delta = _dot(t, wp2[...]) + bp2r[...]
        a = a1d - a1s + _dot(delta, wa1[...]) + ba1r[...]
        a = jnp.maximum(_ln(a, gar[...], bar[...]), 0.0)
        alpha = _dot(a, wa2[...]) + ba2r[...]
        ex = jnp.exp(alpha)
        ex_ref[...] = ex
        msg_ref[...] = ex * (h + delta)

    full = lambda s: pl.BlockSpec(s, lambda i: (0, 0))
    return pl.pallas_call(
        body,
        grid=(grid,),
        in_specs=[
            pl.BlockSpec((blk, 128), lambda i: (i, 0)),
            pl.BlockSpec((blk, 128), lambda i: (i, 0)),
            full((16, 64)), full((1, 64)), full((1, 64)), full((1, 64)),
            full((64, 128)), full((1, 128)),
            full((128, 64)), full((1, 64)), full((1, 64)), full((1, 64)),
            full((64, 128)), full((1, 128)),
        ],
        out_specs=[
            pl.BlockSpec((blk, 128), lambda i: (i, 0)),
            pl.BlockSpec((blk, 128), lambda i: (i, 0)),
        ],
        out_shape=[
            jax.ShapeDtypeStruct((epad, 128), F32),
            jax.ShapeDtypeStruct((epad, 128), F32),
        ],
    )(gsrc, gdst, Wp1T, bp1, gp, betap, Wp2T, bp2, Wa1T, ba1, ga, betaa,
      Wa2T, ba2)


def _final(num, den, WoT, bo2):
    npad, d = num.shape
    blk = 1280
    grid = npad // blk

    def body(n_ref, d_ref, wo, bo, out_ref):
        r = n_ref[...] / (d_ref[...] + 1e-16)
        out_ref[...] = jnp.maximum(_dot(r, wo[...]) + bo[...], 0.0)

    full = lambda s: pl.BlockSpec(s, lambda i: (0, 0))
    return pl.pallas_call(
        body,
        grid=(grid,),
        in_specs=[
            pl.BlockSpec((blk, d), lambda i: (i, 0)),
            pl.BlockSpec((blk, d), lambda i: (i, 0)),
            full((d, d)), full((1, d)),
        ],
        out_specs=pl.BlockSpec((blk, d), lambda i: (i, 0)),
        out_shape=jax.ShapeDtypeStruct((npad, d), F32),
    )(num, den, WoT, bo2)


def _sc_gather(src_ids, dst_ids, src_tab, dst_tab):
    epad = src_ids.shape[0]
    wd = dst_tab.shape[1]
    bpw = epad // NW
    nch = bpw // CG
    mesh = plsc.VectorSubcoreMesh(core_axis_name="c", subcore_axis_name="s")

    assert nch % 2 == 1 and nch >= 3, nch
    npairs = (nch - 1) // 2

    @functools.partial(
        pl.kernel,
        out_type=(
            jax.ShapeDtypeStruct((epad, 128), jnp.int32),
            jax.ShapeDtypeStruct((epad, wd), F32),
        ),
        mesh=mesh,
        scratch_types=[
            [pltpu.VMEM((CG,), jnp.int32)] * 2,
            [pltpu.VMEM((CG,), jnp.int32)] * 2,
            [pltpu.VMEM((CG, 128), jnp.int32)] * 2,
            [pltpu.VMEM((CG, wd), F32)] * 2,
            [pltpu.SemaphoreType.DMA] * 2,
            [pltpu.SemaphoreType.DMA] * 2,
        ],
    )
    def k(sid_h, did_h, stab_h, dtab_h, gsrc_h, gdst_h, idx_s, idx_d, buf_s,
          buf_d, sem_g, sem_w):
        wid = lax.axis_index("s") * SC_CORES + lax.axis_index("c")
        base = wid * bpw

        def g_start(off, t):
            pltpu.sync_copy(sid_h.at[pl.ds(off, CG)], idx_s[t])
            pltpu.sync_copy(did_h.at[pl.ds(off, CG)], idx_d[t])
            pltpu.async_copy(stab_h.at[idx_s[t]], buf_s[t], sem_g[t])
            pltpu.async_copy(dtab_h.at[idx_d[t]], buf_d[t], sem_g[t])

        def g_wait(t):
            pltpu.make_async_copy(stab_h.at[idx_s[t]], buf_s[t], sem_g[t]).wait()
            pltpu.make_async_copy(dtab_h.at[idx_d[t]], buf_d[t], sem_g[t]).wait()

        def w_start(off, t):
            pltpu.async_copy(buf_s[t], gsrc_h.at[pl.ds(off, CG)], sem_w[t])
            pltpu.async_copy(buf_d[t], gdst_h.at[pl.ds(off, CG)], sem_w[t])

        def w_wait(off, t):
            pltpu.make_async_copy(buf_s[t], gsrc_h.at[pl.ds(off, CG)], sem_w[t]).wait()
            pltpu.make_async_copy(buf_d[t], gdst_h.at[pl.ds(off, CG)], sem_w[t]).wait()

        g_start(base, 0)

        def body(p, carry):
            offa = base + (2 * p) * CG
            offb = offa + CG
            offn = offb + CG
            g_wait(0)
            pl.when(p > 0)(lambda: w_wait(offa - CG, 1))
            g_start(offb, 1)
            w_start(offa, 0)
            g_wait(1)
            w_start(offb, 1)
            w_wait(offa, 0)
            g_start(offn, 0)
            return carry

        lax.fori_loop(0, npairs, body, 0)
        last = base + (nch - 1) * CG
        g_wait(0)
        w_wait(last - CG, 1)
        w_start(last, 0)
        w_wait(last, 0)

    return k(src_ids, dst_ids, src_tab, dst_tab)


def _sc_scatter(msg, ex, dst_ids, npad):
    epad, d = msg.shape
    per_tile = epad // SC_SUBCORES
    nch = per_tile // CS
    rows_per_tile = npad // SC_SUBCORES
    mesh = plsc.VectorSubcoreMesh(core_axis_name="c", subcore_axis_name="s")

    @functools.partial(
        pl.kernel,
        out_type=(
            jax.ShapeDtypeStruct((npad, d), F32),
            jax.ShapeDtypeStruct((npad, d), F32),
        ),
        mesh=mesh,
        scratch_types=[
            [pltpu.VMEM((CS, d), F32)] * 2,
            [pltpu.VMEM((CS,), jnp.int32)] * 2,
            pltpu.VMEM((64, d), F32),
            pltpu.VMEM_SHARED((npad, d), F32),
            [pltpu.SemaphoreType.DMA] * 2,
            [pltpu.SemaphoreType.DMA] * 2,
        ],
    )
    def k(msg_h, ex_h, did_h, num_h, den_h, rowbuf, idxb, zbuf, acc, sem_l,
          sem_a):
        c = lax.axis_index("c")
        s = lax.axis_index("s")

        for j in range(64):
            for t in range(d // 16):
                zbuf[j, pl.ds(t * 16, 16)] = jnp.zeros((16,), F32)

        def zero_body(i, carry):
            pltpu.sync_copy(zbuf, acc.at[pl.ds(s * rows_per_tile + i * 64, 64)])
            return carry

        lax.fori_loop(0, rows_per_tile // 64, zero_body, 0)
        plsc.subcore_barrier()

        def process(tbl_h):
            base = s * per_tile

            def l_start(off, t):
                pltpu.sync_copy(did_h.at[pl.ds(off, CS)], idxb[t])
                pltpu.async_copy(tbl_h.at[pl.ds(off, CS)], rowbuf[t], sem_l[t])

            def l_wait(off, t):
                pltpu.make_async_copy(
                    tbl_h.at[pl.ds(off, CS)], rowbuf[t], sem_l[t]).wait()

            def a_start(t):
                pltpu.async_copy(rowbuf[t], acc.at[idxb[t]], sem_a[t], add=True)

            def a_wait(t):
                pltpu.make_async_copy(
                    rowbuf[t], acc.at[idxb[t]], sem_a[t]).wait()

            l_start(base, 0)

            def body(p, carry):
                offa = base + (2 * p) * CS
                offb = offa + CS
                offn = offb + CS
                pl.when(p > 0)(lambda: a_wait(1))
                l_start(offb, 1)
                l_wait(offa, 0)
                a_start(0)
                l_wait(offb, 1)
                a_wait(0)
                pl.when(p + 1 < nch // 2)(lambda: l_start(offn, 0))
                a_start(1)
                return carry

            lax.fori_loop(0, nch // 2, body, 0)
            a_wait(1)

        pl.when(c == 0)(lambda: process(msg_h))
        pl.when(c == 1)(lambda: process(ex_h))
        plsc.subcore_barrier()

        def dump(out_h):
            pltpu.sync_copy(
                acc.at[pl.ds(s * rows_per_tile, rows_per_tile)],
                out_h.at[pl.ds(s * rows_per_tile, rows_per_tile)])

        pl.when(c == 0)(lambda: dump(num_h))
        pl.when(c == 1)(lambda: dump(den_h))

    return k(msg, ex, dst_ids)


def kernel(x, pos, edge_index, Wi, bi, Wo, bo, Wlin, Wsrc, Wdst, Wp1, bp1,
           gp, betap, Wp2, bp2, Wa1, ba1, ga, betaa, Wa2, ba2):
    n, dm = x.shape
    e = edge_index.shape[1]
    etot = e + n

    npad = ((n + 1 + 255) // 256) * 256
    unit = NW * CG
    epad = ((etot + unit - 1) // unit) * unit

    loop = jnp.arange(n, dtype=jnp.int32)
    pad_e = epad - etot
    src_full = jnp.concatenate([
        edge_index[0].astype(jnp.int32), loop,
        jnp.zeros((pad_e,), jnp.int32)])
    dst_full = jnp.concatenate([
        edge_index[1].astype(jnp.int32), loop,
        n + (jnp.arange(pad_e, dtype=jnp.int32) % (npad - n))])

    xp = jnp.pad(x, ((0, npad - n), (0, 0)))
    posp = jnp.pad(pos, ((0, npad - n), (0, 16 - pos.shape[1])))
    WiT = Wi.T
    WlinT = Wlin.T
    WsaT = (Wa1 @ Wsrc).T
    WdaT = (Wa1 @ Wdst).T
    Wp1T = jnp.pad(Wp1.T, ((0, 16 - Wp1.shape[1]), (0, 0)))
    Wp2T = Wp2.T
    Wa1T = Wa1.T
    Wa2T = Wa2.T
    WoT = Wo.T
    r1 = lambda v: v.reshape(1, -1)

    srctab, dsttab = _node_proj(xp, posp, WiT, r1(bi), WlinT, WsaT, WdaT)
    gsrc, gdst = _sc_gather(src_full, dst_full, srctab, dsttab)
    msg, ex = _edge_mlp(gsrc, gdst, Wp1T, r1(bp1), r1(gp), r1(betap), Wp2T,
                        r1(bp2), Wa1T, r1(ba1), r1(ga), r1(betaa), Wa2T,
                        r1(ba2))
    num, den = _sc_scatter(msg, ex, dst_full, npad)
    out = _final(num, den, WoT, r1(bo))
    return out[:n]


# speedup vs baseline: 8.2206x; 1.0411x over previous
import functools

import jax
import jax.numpy as jnp
from jax import lax
from jax.experimental import pallas as pl
from jax.experimental.pallas import tpu as pltpu
from jax.experimental.pallas import tpu_sc as plsc

F32 = jnp.float32
BF16 = jnp.bfloat16

SC_CORES = 2
SC_SUBCORES = 16
NW = SC_CORES * SC_SUBCORES
CG = 128
CS = 128


def _ln(v, g, b):
    mu = jnp.mean(v, axis=-1, keepdims=True)
    var = jnp.mean((v - mu) ** 2, axis=-1, keepdims=True)
    return (v - mu) * lax.rsqrt(var + 1e-5) * g + b


def _ln_mm(v, g, b, m):
    mu = _dot(v, m)
    d = v - mu
    var = _dot(d * d, m)
    return d * lax.rsqrt(var + 1e-5) * g + b


def _dot(a, b):
    return jnp.dot(a, b, preferred_element_type=F32)


def _pack2(a, b):
    ua = (lax.bitcast_convert_type(a, jnp.uint32) + jnp.uint32(0x8000)) >> 16
    ub = (lax.bitcast_convert_type(b, jnp.uint32) + jnp.uint32(0x8000)) & jnp.uint32(0xFFFF0000)
    return lax.bitcast_convert_type(ua | ub, jnp.int32)


def _unpack2(u):
    uu = lax.bitcast_convert_type(u, jnp.uint32)
    a = lax.bitcast_convert_type(uu << 16, F32)
    b = lax.bitcast_convert_type(uu & jnp.uint32(0xFFFF0000), F32)
    return a, b


def _node_proj(xp, posp, WiT, bi2, WlinT, WsaT, WdaT, Wp1T16):
    npad, d = xp.shape
    blk = 1280
    grid = npad // blk

    def body(x_ref, pos_ref, wi, bi, wlin, wsa, wda, wp1, src_ref, dst_ref):
        x1 = jnp.maximum(_dot(x_ref[...], wi[...]) + bi[...], 0.0)
        h = _dot(x1, wlin[...])
        a1s = _dot(x1, wsa[...])
        psv = pos_ref[...]
        src_ref[:, 0:64] = _pack2(h[:, 0:64], h[:, 64:128])
        src_ref[:, 64:96] = _pack2(a1s[:, 0:32], a1s[:, 32:64])
        src_ref[:, 96:104] = _pack2(psv[:, 0:8], psv[:, 8:16])
        src_ref[:, 104:128] = jnp.zeros((blk, 24), jnp.int32)
        dst_ref[:, 0:64] = _dot(x1, wda[...])
        dst_ref[:, 64:80] = psv
        dst_ref[:, 80:128] = jnp.zeros((blk, 48), F32)

    full = lambda s: pl.BlockSpec(s, lambda i: (0, 0))
    return pl.pallas_call(
        body,
        grid=(grid,),
        in_specs=[
            pl.BlockSpec((blk, d), lambda i: (i, 0)),
            pl.BlockSpec((blk, 16), lambda i: (i, 0)),
            full((d, d)), full((1, d)), full((d, d)), full((d, 64)), full((d, 64)),
            full((16, 64)),
        ],
        out_specs=[
            pl.BlockSpec((blk, 128), lambda i: (i, 0)),
            pl.BlockSpec((blk, 128), lambda i: (i, 0)),
        ],
        out_shape=[
            jax.ShapeDtypeStruct((npad, 128), jnp.int32),
            jax.ShapeDtypeStruct((npad, 128), F32),
        ],
    )(xp, posp, WiT, bi2, WlinT, WsaT, WdaT, Wp1T16)


def _edge_mlp(gsrc, gdst, Wp1pad, Wp1a, bp1, gp, betap, Wp2T, bp2, Wa1T,
              ba1, ga, betaa, Wa2T, ba2, M64):
    epad = gsrc.shape[0]
    blk = 1024
    grid = epad // blk

    def body(gs_ref, gd_ref, wp1p, wp1a, bp1r, gpr, bpr, wp2, bp2r, wa1,
             ba1r, gar, bar, wa2, ba2r, m64, msg_ref, ex_ref):
        h0, h1 = _unpack2(gs_ref[:, 0:64])
        s0, s1 = _unpack2(gs_ref[:, 64:96])
        p0, _ = _unpack2(gs_ref[:, 96:104])
        gd = gd_ref[...]
        a1d = gd_ref[:, 0:64]
        t = _dot(gd, wp1p[...]) - _dot(p0, wp1a[...]) + bp1r[...]
        t = jnp.maximum(_ln_mm(t, gpr[...], bpr[...], m64[...]), 0.0)
        delta = _dot(t, wp2[...]) + bp2r[...]
        a1s = jnp.concatenate([s0, s1], axis=-1)
        a = a1d - a1s + _dot(delta, wa1[...]) + ba1r[...]
        a = jnp.maximum(_ln_mm(a, gar[...], bar[...], m64[...]), 0.0)
        alpha = _dot(a, wa2[...]) + ba2r[...]
        ex = jnp.exp(alpha)
        ex_ref[...] = ex
        msg_ref[:, 0:64] = ex[:, 0:64] * (h0 + delta[:, 0:64])
        msg_ref[:, 64:128] = ex[:, 64:128] * (h1 + delta[:, 64:128])

    full = lambda s: pl.BlockSpec(s, lambda i: (0, 0))
    return pl.pallas_call(
        body,
        grid=(grid,),
        in_specs=[
            pl.BlockSpec((blk, 128), lambda i: (i, 0)),
            pl.BlockSpec((blk, 128), lambda i: (i, 0)),
            full((128, 64)), full((8, 64)),
            full((1, 64)), full((1, 64)), full((1, 64)),
            full((64, 128)), full((1, 128)),
            full((128, 64)), full((1, 64)), full((1, 64)), full((1, 64)),
            full((64, 128)), full((1, 128)), full((64, 64)),
        ],
        out_specs=[
            pl.BlockSpec((blk, 128), lambda i: (i, 0)),
            pl.BlockSpec((blk, 128), lambda i: (i, 0)),
        ],
        out_shape=[
            jax.ShapeDtypeStruct((epad, 128), F32),
            jax.ShapeDtypeStruct((epad, 128), F32),
        ],
    )(gsrc, gdst, Wp1pad, Wp1a, bp1, gp, betap, Wp2T, bp2, Wa1T, ba1, ga,
      betaa, Wa2T, ba2, M64)


def _final(num, den, WoT, bo2):
    npad, d = num.shape
    blk = 1280
    grid = npad // blk

    def body(n_ref, d_ref, wo, bo, out_ref):
        r = n_ref[...] / (d_ref[...] + 1e-16)
        out_ref[...] = jnp.maximum(_dot(r, wo[...]) + bo[...], 0.0)

    full = lambda s: pl.BlockSpec(s, lambda i: (0, 0))
    return pl.pallas_call(
        body,
        grid=(grid,),
        in_specs=[
            pl.BlockSpec((blk, d), lambda i: (i, 0)),
            pl.BlockSpec((blk, d), lambda i: (i, 0)),
            full((d, d)), full((1, d)),
        ],
        out_specs=pl.BlockSpec((blk, d), lambda i: (i, 0)),
        out_shape=jax.ShapeDtypeStruct((npad, d), F32),
    )(num, den, WoT, bo2)


def _sc_gather(src_ids, dst_ids, src_tab, dst_tab):
    epad = src_ids.shape[0]
    wd = dst_tab.shape[1]
    bpw = epad // NW
    nch = bpw // CG
    mesh = plsc.VectorSubcoreMesh(core_axis_name="c", subcore_axis_name="s")

    assert nch % 2 == 1 and nch >= 3, nch
    npairs = (nch - 1) // 2

    @functools.partial(
        pl.kernel,
        out_type=(
            jax.ShapeDtypeStruct((epad, 128), jnp.int32),
            jax.ShapeDtypeStruct((epad, wd), F32),
        ),
        mesh=mesh,
        scratch_types=[
            [pltpu.VMEM((CG,), jnp.int32)] * 2,
            [pltpu.VMEM((CG,), jnp.int32)] * 2,
            [pltpu.VMEM((CG, 128), jnp.int32)] * 2,
            [pltpu.VMEM((CG, wd), F32)] * 2,
            [pltpu.SemaphoreType.DMA] * 2,
            [pltpu.SemaphoreType.DMA] * 2,
        ],
    )
    def k(sid_h, did_h, stab_h, dtab_h, gsrc_h, gdst_h, idx_s, idx_d, buf_s,
          buf_d, sem_g, sem_w):
        wid = lax.axis_index("s") * SC_CORES + lax.axis_index("c")
        base = wid * bpw

        def g_start(off, t):
            pltpu.sync_copy(sid_h.at[pl.ds(off, CG)], idx_s[t])
            pltpu.sync_copy(did_h.at[pl.ds(off, CG)], idx_d[t])
            pltpu.async_copy(stab_h.at[idx_s[t]], buf_s[t], sem_g[t])
            pltpu.async_copy(dtab_h.at[idx_d[t]], buf_d[t], sem_g[t])

        def g_wait(t):
            pltpu.make_async_copy(stab_h.at[idx_s[t]], buf_s[t], sem_g[t]).wait()
            pltpu.make_async_copy(dtab_h.at[idx_d[t]], buf_d[t], sem_g[t]).wait()

        def w_start(off, t):
            pltpu.async_copy(buf_s[t], gsrc_h.at[pl.ds(off, CG)], sem_w[t])
            pltpu.async_copy(buf_d[t], gdst_h.at[pl.ds(off, CG)], sem_w[t])

        def w_wait(off, t):
            pltpu.make_async_copy(buf_s[t], gsrc_h.at[pl.ds(off, CG)], sem_w[t]).wait()
            pltpu.make_async_copy(buf_d[t], gdst_h.at[pl.ds(off, CG)], sem_w[t]).wait()

        g_start(base, 0)

        def body(p, carry):
            offa = base + (2 * p) * CG
            offb = offa + CG
            offn = offb + CG
            g_wait(0)
            pl.when(p > 0)(lambda: w_wait(offa - CG, 1))
            g_start(offb, 1)
            w_start(offa, 0)
            g_wait(1)
            w_start(offb, 1)
            w_wait(offa, 0)
            g_start(offn, 0)
            return carry

        lax.fori_loop(0, npairs, body, 0)
        last = base + (nch - 1) * CG
        g_wait(0)
        w_wait(last - CG, 1)
        w_start(last, 0)
        w_wait(last, 0)

    return k(src_ids, dst_ids, src_tab, dst_tab)


def _sc_scatter(msg, ex, dst_ids, npad):
    epad, d = msg.shape
    per_tile = epad // SC_SUBCORES
    nch = per_tile // CS
    rows_per_tile = npad // SC_SUBCORES
    mesh = plsc.VectorSubcoreMesh(core_axis_name="c", subcore_axis_name="s")

    @functools.partial(
        pl.kernel,
        out_type=(
            jax.ShapeDtypeStruct((npad, d), F32),
            jax.ShapeDtypeStruct((npad, d), F32),
        ),
        mesh=mesh,
        scratch_types=[
            [pltpu.VMEM((CS, d), F32)] * 2,
            [pltpu.VMEM((CS,), jnp.int32)] * 2,
            pltpu.VMEM((64, d), F32),
            pltpu.VMEM_SHARED((npad, d), F32),
            [pltpu.SemaphoreType.DMA] * 2,
            [pltpu.SemaphoreType.DMA] * 2,
        ],
    )
    def k(msg_h, ex_h, did_h, num_h, den_h, rowbuf, idxb, zbuf, acc, sem_l,
          sem_a):
        c = lax.axis_index("c")
        s = lax.axis_index("s")

        for j in range(64):
            for t in range(d // 16):
                zbuf[j, pl.ds(t * 16, 16)] = jnp.zeros((16,), F32)

        def zero_body(i, carry):
            pltpu.sync_copy(zbuf, acc.at[pl.ds(s * rows_per_tile + i * 64, 64)])
            return carry

        lax.fori_loop(0, rows_per_tile // 64, zero_body, 0)
        plsc.subcore_barrier()

        def process(tbl_h):
            base = s * per_tile

            def l_start(off, t):
                pltpu.sync_copy(did_h.at[pl.ds(off, CS)], idxb[t])
                pltpu.async_copy(tbl_h.at[pl.ds(off, CS)], rowbuf[t], sem_l[t])

            def l_wait(off, t):
                pltpu.make_async_copy(
                    tbl_h.at[pl.ds(off, CS)], rowbuf[t], sem_l[t]).wait()

            def a_start(t):
                pltpu.async_copy(rowbuf[t], acc.at[idxb[t]], sem_a[t], add=True)

            def a_wait(t):
                pltpu.make_async_copy(
                    rowbuf[t], acc.at[idxb[t]], sem_a[t]).wait()

            l_start(base, 0)

            def body(p, carry):
                offa = base + (2 * p) * CS
                offb = offa + CS
                offn = offb + CS
                pl.when(p > 0)(lambda: a_wait(1))
                l_start(offb, 1)
                l_wait(offa, 0)
                a_start(0)
                l_wait(offb, 1)
                a_wait(0)
                pl.when(p + 1 < nch // 2)(lambda: l_start(offn, 0))
                a_start(1)
                return carry

            lax.fori_loop(0, nch // 2, body, 0)
            a_wait(1)

        pl.when(c == 0)(lambda: process(msg_h))
        pl.when(c == 1)(lambda: process(ex_h))
        plsc.subcore_barrier()

        def dump(out_h):
            pltpu.sync_copy(
                acc.at[pl.ds(s * rows_per_tile, rows_per_tile)],
                out_h.at[pl.ds(s * rows_per_tile, rows_per_tile)])

        pl.when(c == 0)(lambda: dump(num_h))
        pl.when(c == 1)(lambda: dump(den_h))

    return k(msg, ex, dst_ids)


def kernel(x, pos, edge_index, Wi, bi, Wo, bo, Wlin, Wsrc, Wdst, Wp1, bp1,
           gp, betap, Wp2, bp2, Wa1, ba1, ga, betaa, Wa2, ba2):
    n, dm = x.shape
    e = edge_index.shape[1]
    etot = e + n

    npad = ((n + 1 + 255) // 256) * 256
    unit = NW * CG
    epad = ((etot + unit - 1) // unit) * unit

    loop = jnp.arange(n, dtype=jnp.int32)
    pad_e = epad - etot
    src_full = jnp.concatenate([
        edge_index[0].astype(jnp.int32), loop,
        jnp.zeros((pad_e,), jnp.int32)])
    dst_full = jnp.concatenate([
        edge_index[1].astype(jnp.int32), loop,
        n + (jnp.arange(pad_e, dtype=jnp.int32) % (npad - n))])

    xp = jnp.pad(x, ((0, npad - n), (0, 0)))
    posp = jnp.pad(pos, ((0, npad - n), (0, 16 - pos.shape[1])))
    WiT = Wi.T
    WlinT = Wlin.T
    WsaT = (Wa1 @ Wsrc).T
    WdaT = (Wa1 @ Wdst).T
    Wp1T16 = jnp.pad(Wp1.T, ((0, 16 - Wp1.shape[1]), (0, 0)))
    Wp1a = jnp.pad(Wp1.T, ((0, 8 - Wp1.shape[1]), (0, 0)))
    Wp1pad = jnp.zeros((128, 64), F32).at[64:64 + Wp1.shape[1]].set(Wp1.T)
    M64 = jnp.full((64, 64), 1.0 / 64.0, F32)
    Wp2T = Wp2.T
    Wa1T = Wa1.T
    Wa2T = Wa2.T
    WoT = Wo.T
    r1 = lambda v: v.reshape(1, -1)

    srctab, dsttab = _node_proj(xp, posp, WiT, r1(bi), WlinT, WsaT, WdaT,
                                Wp1T16)
    gsrc, gdst = _sc_gather(src_full, dst_full, srctab, dsttab)
    msg, ex = _edge_mlp(gsrc, gdst, Wp1pad, Wp1a, r1(bp1), r1(gp),
                        r1(betap), Wp2T, r1(bp2), Wa1T, r1(ba1), r1(ga),
                        r1(betaa), Wa2T, r1(ba2), M64)
    num, den = _sc_scatter(msg, ex, dst_full, npad)
    out = _final(num, den, WoT, r1(bo))
    return out[:n]


# speedup vs baseline: 10.4727x; 1.2740x over previous
import functools

import jax
import jax.numpy as jnp
from jax import lax
from jax.experimental import pallas as pl
from jax.experimental.pallas import tpu as pltpu
from jax.experimental.pallas import tpu_sc as plsc

F32 = jnp.float32
BF16 = jnp.bfloat16

SC_CORES = 2
SC_SUBCORES = 16
NW = SC_CORES * SC_SUBCORES
CG = 128
CS = 128


def _ln(v, g, b):
    mu = jnp.mean(v, axis=-1, keepdims=True)
    var = jnp.mean((v - mu) ** 2, axis=-1, keepdims=True)
    return (v - mu) * lax.rsqrt(var + 1e-5) * g + b


def _ln_mm(v, g, b, m):
    mu = _dot(v, m)
    d = v - mu
    var = _dot(d * d, m)
    return d * lax.rsqrt(var + 1e-5) * g + b


def _dot(a, b):
    return jnp.dot(a, b, preferred_element_type=F32)


def _pack2(a, b):
    ua = (lax.bitcast_convert_type(a, jnp.uint32) + jnp.uint32(0x8000)) >> 16
    ub = (lax.bitcast_convert_type(b, jnp.uint32) + jnp.uint32(0x8000)) & jnp.uint32(0xFFFF0000)
    return lax.bitcast_convert_type(ua | ub, jnp.int32)


def _unpack2(u):
    uu = lax.bitcast_convert_type(u, jnp.uint32)
    a = lax.bitcast_convert_type(uu << 16, F32)
    b = lax.bitcast_convert_type(uu & jnp.uint32(0xFFFF0000), F32)
    return a, b


def _node_proj(xp, posp, WiT, bi2, WlinT, WsaT, WdaT, Wp1T16):
    npad, d = xp.shape
    blk = 1280
    grid = npad // blk

    def body(x_ref, pos_ref, wi, bi, wlin, wsa, wda, wp1, src_ref, dst_ref):
        x1 = jnp.maximum(_dot(x_ref[...], wi[...]) + bi[...], 0.0)
        h = _dot(x1, wlin[...])
        a1s = _dot(x1, wsa[...])
        psv = pos_ref[...]
        src_ref[:, 0:64] = _pack2(h[:, 0:64], h[:, 64:128])
        src_ref[:, 64:96] = _pack2(a1s[:, 0:32], a1s[:, 32:64])
        src_ref[:, 96:104] = _pack2(psv[:, 0:8], psv[:, 8:16])
        src_ref[:, 104:128] = jnp.zeros((blk, 24), jnp.int32)
        dst_ref[:, 0:64] = _dot(x1, wda[...])
        dst_ref[:, 64:80] = psv
        dst_ref[:, 80:128] = jnp.zeros((blk, 48), F32)

    full = lambda s: pl.BlockSpec(s, lambda i: (0, 0))
    return pl.pallas_call(
        body,
        grid=(grid,),
        in_specs=[
            pl.BlockSpec((blk, d), lambda i: (i, 0)),
            pl.BlockSpec((blk, 16), lambda i: (i, 0)),
            full((d, d)), full((1, d)), full((d, d)), full((d, 64)), full((d, 64)),
            full((16, 64)),
        ],
        out_specs=[
            pl.BlockSpec((blk, 128), lambda i: (i, 0)),
            pl.BlockSpec((blk, 128), lambda i: (i, 0)),
        ],
        out_shape=[
            jax.ShapeDtypeStruct((npad, 128), jnp.int32),
            jax.ShapeDtypeStruct((npad, 128), F32),
        ],
    )(xp, posp, WiT, bi2, WlinT, WsaT, WdaT, Wp1T16)


def _edge_mlp(gsrc, gdst, Wp1pad, Wp1a, bp1, gp, betap, Wp2T, bp2, Wa1T,
              ba1, ga, betaa, Wa2T, ba2, M64):
    epad = gsrc.shape[0]
    blk = 1024
    grid = epad // blk

    def body(gs_ref, gd_ref, wp1p, wp1a, bp1r, gpr, bpr, wp2, bp2r, wa1,
             ba1r, gar, bar, wa2, ba2r, m64, msg_ref, ex_ref):
        h0, h1 = _unpack2(gs_ref[:, 0:64])
        s0, s1 = _unpack2(gs_ref[:, 64:96])
        p0, _ = _unpack2(gs_ref[:, 96:104])
        gd = gd_ref[...]
        a1d = gd_ref[:, 0:64]
        t = _dot(gd, wp1p[...]) - _dot(p0, wp1a[...]) + bp1r[...]
        t = jnp.maximum(_ln_mm(t, gpr[...], bpr[...], m64[...]), 0.0)
        delta = _dot(t, wp2[...]) + bp2r[...]
        a1s = jnp.concatenate([s0, s1], axis=-1)
        a = a1d - a1s + _dot(delta, wa1[...]) + ba1r[...]
        a = jnp.maximum(_ln_mm(a, gar[...], bar[...], m64[...]), 0.0)
        alpha = _dot(a, wa2[...]) + ba2r[...]
        ex = jnp.exp(alpha)
        ex_ref[...] = ex
        msg_ref[:, 0:64] = ex[:, 0:64] * (h0 + delta[:, 0:64])
        msg_ref[:, 64:128] = ex[:, 64:128] * (h1 + delta[:, 64:128])

    full = lambda s: pl.BlockSpec(s, lambda i: (0, 0))
    return pl.pallas_call(
        body,
        grid=(grid,),
        in_specs=[
            pl.BlockSpec((blk, 128), lambda i: (i, 0)),
            pl.BlockSpec((blk, 128), lambda i: (i, 0)),
            full((128, 64)), full((8, 64)),
            full((1, 64)), full((1, 64)), full((1, 64)),
            full((64, 128)), full((1, 128)),
            full((128, 64)), full((1, 64)), full((1, 64)), full((1, 64)),
            full((64, 128)), full((1, 128)), full((64, 64)),
        ],
        out_specs=[
            pl.BlockSpec((blk, 128), lambda i: (i, 0)),
            pl.BlockSpec((blk, 128), lambda i: (i, 0)),
        ],
        out_shape=[
            jax.ShapeDtypeStruct((epad, 128), F32),
            jax.ShapeDtypeStruct((epad, 128), F32),
        ],
    )(gsrc, gdst, Wp1pad, Wp1a, bp1, gp, betap, Wp2T, bp2, Wa1T, ba1, ga,
      betaa, Wa2T, ba2, M64)


def _final(nums, dens, WoT, bo2):
    npad, d = nums[0].shape
    blk = 1280
    grid = npad // blk

    def body(n1, n2, n3, d1, d2, d3, wo, bo, out_ref):
        num = n1[...] + n2[...] + n3[...]
        den = d1[...] + d2[...] + d3[...]
        r = num / (den + 1e-16)
        out_ref[...] = jnp.maximum(_dot(r, wo[...]) + bo[...], 0.0)

    full = lambda s: pl.BlockSpec(s, lambda i: (0, 0))
    row = pl.BlockSpec((blk, d), lambda i: (i, 0))
    return pl.pallas_call(
        body,
        grid=(grid,),
        in_specs=[row, row, row, row, row, row, full((d, d)), full((1, d))],
        out_specs=row,
        out_shape=jax.ShapeDtypeStruct((npad, d), F32),
    )(*nums, *dens, WoT, bo2)


def _sc_gather(src_ids, dst_ids, src_tab, dst_tab):
    epad = src_ids.shape[0]
    wd = dst_tab.shape[1]
    bpw = epad // NW
    nch = bpw // CG
    mesh = plsc.VectorSubcoreMesh(core_axis_name="c", subcore_axis_name="s")

    assert nch % 2 == 1 and nch >= 3, nch
    npairs = (nch - 1) // 2
    assert epad % 1024 == 0

    @functools.partial(
        pl.kernel,
        out_type=(
            jax.ShapeDtypeStruct((epad, 128), jnp.int32),
            jax.ShapeDtypeStruct((epad, wd), F32),
        ),
        mesh=mesh,
        scratch_types=[
            [pltpu.VMEM((CG,), jnp.int32)] * 2,
            [pltpu.VMEM((CG,), jnp.int32)] * 2,
            [pltpu.VMEM((CG, 128), jnp.int32)] * 2,
            [pltpu.VMEM((CG, wd), F32)] * 2,
            [pltpu.SemaphoreType.DMA] * 2,
            [pltpu.SemaphoreType.DMA] * 2,
        ],
    )
    def k(sid_h, did_h, stab_h, dtab_h, gsrc_h, gdst_h, idx_s, idx_d, buf_s,
          buf_d, sem_g, sem_w):
        wid = lax.axis_index("s") * SC_CORES + lax.axis_index("c")
        base = wid * bpw

        def g_start(off, t):
            pltpu.sync_copy(sid_h.at[pl.ds(off, CG)], idx_s[t])
            pltpu.sync_copy(did_h.at[pl.ds(off, CG)], idx_d[t])
            pltpu.async_copy(stab_h.at[idx_s[t]], buf_s[t], sem_g[t])
            pltpu.async_copy(dtab_h.at[idx_d[t]], buf_d[t], sem_g[t])

        def g_wait(t):
            pltpu.make_async_copy(stab_h.at[idx_s[t]], buf_s[t], sem_g[t]).wait()
            pltpu.make_async_copy(dtab_h.at[idx_d[t]], buf_d[t], sem_g[t]).wait()

        def w_start(off, t):
            pltpu.async_copy(buf_s[t], gsrc_h.at[pl.ds(off, CG)], sem_w[t])
            pltpu.async_copy(buf_d[t], gdst_h.at[pl.ds(off, CG)], sem_w[t])

        def w_wait(off, t):
            pltpu.make_async_copy(buf_s[t], gsrc_h.at[pl.ds(off, CG)], sem_w[t]).wait()
            pltpu.make_async_copy(buf_d[t], gdst_h.at[pl.ds(off, CG)], sem_w[t]).wait()

        g_start(base, 0)

        def body(p, carry):
            offa = base + (2 * p) * CG
            offb = offa + CG
            offn = offb + CG
            g_wait(0)
            pl.when(p > 0)(lambda: w_wait(offa - CG, 1))
            g_start(offb, 1)
            w_start(offa, 0)
            g_wait(1)
            w_start(offb, 1)
            w_wait(offa, 0)
            g_start(offn, 0)
            return carry

        lax.fori_loop(0, npairs, body, 0)
        last = base + (nch - 1) * CG
        g_wait(0)
        w_wait(last - CG, 1)
        w_start(last, 0)
        w_wait(last, 0)

    return k(src_ids, dst_ids, src_tab, dst_tab)


def _sc_scatter(msg, ex, dst_ids, npad):
    epad, d = msg.shape
    per_tile = epad // SC_SUBCORES
    nch = per_tile // CS
    rows_per_tile = npad // SC_SUBCORES
    mesh = plsc.VectorSubcoreMesh(core_axis_name="c", subcore_axis_name="s")

    @functools.partial(
        pl.kernel,
        out_type=(
            jax.ShapeDtypeStruct((npad, d), F32),
            jax.ShapeDtypeStruct((npad, d), F32),
        ),
        mesh=mesh,
        scratch_types=[
            [pltpu.VMEM((CS, d), F32)] * 2,
            [pltpu.VMEM((CS,), jnp.int32)] * 2,
            pltpu.VMEM((64, d), F32),
            pltpu.VMEM_SHARED((npad, d), F32),
            [pltpu.SemaphoreType.DMA] * 2,
            [pltpu.SemaphoreType.DMA] * 2,
        ],
    )
    def k(msg_h, ex_h, did_h, num_h, den_h, rowbuf, idxb, zbuf, acc, sem_l,
          sem_a):
        c = lax.axis_index("c")
        s = lax.axis_index("s")

        for j in range(64):
            for t in range(d // 16):
                zbuf[j, pl.ds(t * 16, 16)] = jnp.zeros((16,), F32)

        def zero_body(i, carry):
            pltpu.sync_copy(zbuf, acc.at[pl.ds(s * rows_per_tile + i * 64, 64)])
            return carry

        lax.fori_loop(0, rows_per_tile // 64, zero_body, 0)
        plsc.subcore_barrier()

        def process(tbl_h):
            base = s * per_tile

            def l_start(off, t):
                pltpu.sync_copy(did_h.at[pl.ds(off, CS)], idxb[t])
                pltpu.async_copy(tbl_h.at[pl.ds(off, CS)], rowbuf[t], sem_l[t])

            def l_wait(off, t):
                pltpu.make_async_copy(
                    tbl_h.at[pl.ds(off, CS)], rowbuf[t], sem_l[t]).wait()

            def a_start(t):
                pltpu.async_copy(rowbuf[t], acc.at[idxb[t]], sem_a[t], add=True)

            def a_wait(t):
                pltpu.make_async_copy(
                    rowbuf[t], acc.at[idxb[t]], sem_a[t]).wait()

            l_start(base, 0)

            def body(p, carry):
                offa = base + (2 * p) * CS
                offb = offa + CS
                offn = offb + CS
                pl.when(p > 0)(lambda: a_wait(1))
                l_start(offb, 1)
                l_wait(offa, 0)
                a_start(0)
                l_wait(offb, 1)
                a_wait(0)
                pl.when(p + 1 < nch // 2)(lambda: l_start(offn, 0))
                a_start(1)
                return carry

            lax.fori_loop(0, nch // 2, body, 0)
            a_wait(1)

        pl.when(c == 0)(lambda: process(msg_h))
        pl.when(c == 1)(lambda: process(ex_h))
        plsc.subcore_barrier()

        def dump(out_h):
            pltpu.sync_copy(
                acc.at[pl.ds(s * rows_per_tile, rows_per_tile)],
                out_h.at[pl.ds(s * rows_per_tile, rows_per_tile)])

        pl.when(c == 0)(lambda: dump(num_h))
        pl.when(c == 1)(lambda: dump(den_h))

    return k(msg, ex, dst_ids)


def kernel(x, pos, edge_index, Wi, bi, Wo, bo, Wlin, Wsrc, Wdst, Wp1, bp1,
           gp, betap, Wp2, bp2, Wa1, ba1, ga, betaa, Wa2, ba2):
    n, dm = x.shape
    e = edge_index.shape[1]
    etot = e + n

    npad = ((n + 1 + 255) // 256) * 256
    nslc = 3
    unit = NW * CG * nslc
    epad = ((etot + unit - 1) // unit) * unit
    eslc = epad // nslc

    loop = jnp.arange(n, dtype=jnp.int32)
    pad_e = epad - etot
    src_full = jnp.concatenate([
        edge_index[0].astype(jnp.int32), loop,
        jnp.zeros((pad_e,), jnp.int32)])
    dst_full = jnp.concatenate([
        edge_index[1].astype(jnp.int32), loop,
        n + (jnp.arange(pad_e, dtype=jnp.int32) % (npad - n))])

    xp = jnp.pad(x, ((0, npad - n), (0, 0)))
    posp = jnp.pad(pos, ((0, npad - n), (0, 16 - pos.shape[1])))
    WiT = Wi.T
    WlinT = Wlin.T
    WsaT = (Wa1 @ Wsrc).T
    WdaT = (Wa1 @ Wdst).T
    Wp1T16 = jnp.pad(Wp1.T, ((0, 16 - Wp1.shape[1]), (0, 0)))
    Wp1a = jnp.pad(Wp1.T, ((0, 8 - Wp1.shape[1]), (0, 0)))
    Wp1pad = jnp.zeros((128, 64), F32).at[64:64 + Wp1.shape[1]].set(Wp1.T)
    M64 = jnp.full((64, 64), 1.0 / 64.0, F32)
    Wp2T = Wp2.T
    Wa1T = Wa1.T
    Wa2T = Wa2.T
    WoT = Wo.T
    r1 = lambda v: v.reshape(1, -1)

    srctab, dsttab = _node_proj(xp, posp, WiT, r1(bi), WlinT, WsaT, WdaT,
                                Wp1T16)
    nums, dens = [], []
    for k in range(nslc):
        sl = slice(k * eslc, (k + 1) * eslc)
        gsrc, gdst = _sc_gather(src_full[sl], dst_full[sl], srctab, dsttab)
        msg, ex = _edge_mlp(gsrc, gdst, Wp1pad, Wp1a, r1(bp1), r1(gp),
                            r1(betap), Wp2T, r1(bp2), Wa1T, r1(ba1), r1(ga),
                            r1(betaa), Wa2T, r1(ba2), M64)
        num, den = _sc_scatter(msg, ex, dst_full[sl], npad)
        nums.append(num)
        dens.append(den)
    out = _final(nums, dens, WoT, r1(bo))
    return out[:n]


# speedup vs baseline: 10.7725x; 1.0286x over previous
import functools

import jax
import jax.numpy as jnp
from jax import lax
from jax.experimental import pallas as pl
from jax.experimental.pallas import tpu as pltpu
from jax.experimental.pallas import tpu_sc as plsc

F32 = jnp.float32
BF16 = jnp.bfloat16

SC_CORES = 2
SC_SUBCORES = 16
NW = SC_CORES * SC_SUBCORES
CG = 128
CS = 128


def _ln(v, g, b):
    mu = jnp.mean(v, axis=-1, keepdims=True)
    var = jnp.mean((v - mu) ** 2, axis=-1, keepdims=True)
    return (v - mu) * lax.rsqrt(var + 1e-5) * g + b


def _ln_mm(v, g, b, m):
    mu = _dot(v, m)
    d = v - mu
    var = _dot(d * d, m)
    return d * lax.rsqrt(var + 1e-5) * g + b


def _dot(a, b):
    return jnp.dot(a, b, preferred_element_type=F32)


def _pack2(a, b):
    ua = (lax.bitcast_convert_type(a, jnp.uint32) + jnp.uint32(0x8000)) >> 16
    ub = (lax.bitcast_convert_type(b, jnp.uint32) + jnp.uint32(0x8000)) & jnp.uint32(0xFFFF0000)
    return lax.bitcast_convert_type(ua | ub, jnp.int32)


def _unpack2(u):
    uu = lax.bitcast_convert_type(u, jnp.uint32)
    a = lax.bitcast_convert_type(uu << 16, F32)
    b = lax.bitcast_convert_type(uu & jnp.uint32(0xFFFF0000), F32)
    return a, b


def _node_proj(xp, posp, WiT, bi2, WlinT, WsaT, WdaT, Wp1T16):
    npad, d = xp.shape
    blk = 1280
    grid = npad // blk

    def body(x_ref, pos_ref, wi, bi, wlin, wsa, wda, wp1, src_ref, dst_ref):
        x1 = jnp.maximum(_dot(x_ref[...], wi[...]) + bi[...], 0.0)
        h = _dot(x1, wlin[...])
        a1s = _dot(x1, wsa[...])
        psv = pos_ref[...]
        src_ref[:, 0:64] = _pack2(h[:, 0:64], h[:, 64:128])
        src_ref[:, 64:96] = _pack2(a1s[:, 0:32], a1s[:, 32:64])
        src_ref[:, 96:104] = _pack2(psv[:, 0:8], psv[:, 8:16])
        src_ref[:, 104:128] = jnp.zeros((blk, 24), jnp.int32)
        dst_ref[:, 0:64] = _dot(x1, wda[...])
        dst_ref[:, 64:80] = psv
        dst_ref[:, 80:128] = jnp.zeros((blk, 48), F32)

    full = lambda s: pl.BlockSpec(s, lambda i: (0, 0))
    return pl.pallas_call(
        body,
        grid=(grid,),
        in_specs=[
            pl.BlockSpec((blk, d), lambda i: (i, 0)),
            pl.BlockSpec((blk, 16), lambda i: (i, 0)),
            full((d, d)), full((1, d)), full((d, d)), full((d, 64)), full((d, 64)),
            full((16, 64)),
        ],
        out_specs=[
            pl.BlockSpec((blk, 128), lambda i: (i, 0)),
            pl.BlockSpec((blk, 128), lambda i: (i, 0)),
        ],
        out_shape=[
            jax.ShapeDtypeStruct((npad, 128), jnp.int32),
            jax.ShapeDtypeStruct((npad, 128), F32),
        ],
    )(xp, posp, WiT, bi2, WlinT, WsaT, WdaT, Wp1T16)


def _edge_mlp(gsrc, gdst, Wp1pad, Wp1a, bp1, gp, betap, Wp2T, bp2, Wa1T,
              ba1, ga, betaa, Wa2T, ba2, M64):
    epad = gsrc.shape[0]
    blk = 1024
    grid = epad // blk

    def body(gs_ref, gd_ref, wp1p, wp1a, bp1r, gpr, bpr, wp2, bp2r, wa1,
             ba1r, gar, bar, wa2, ba2r, m64, msg_ref, ex_ref):
        h0, h1 = _unpack2(gs_ref[:, 0:64])
        s0, s1 = _unpack2(gs_ref[:, 64:96])
        p0, _ = _unpack2(gs_ref[:, 96:104])
        gd = gd_ref[...]
        a1d = gd_ref[:, 0:64]
        t = _dot(gd, wp1p[...]) - _dot(p0, wp1a[...]) + bp1r[...]
        t = jnp.maximum(_ln_mm(t, gpr[...], bpr[...], m64[...]), 0.0)
        delta = _dot(t, wp2[...]) + bp2r[...]
        a1s = jnp.concatenate([s0, s1], axis=-1)
        a = a1d - a1s + _dot(delta, wa1[...]) + ba1r[...]
        a = jnp.maximum(_ln_mm(a, gar[...], bar[...], m64[...]), 0.0)
        alpha = _dot(a, wa2[...]) + ba2r[...]
        ex = jnp.exp(alpha)
        ex_ref[...] = ex
        msg_ref[:, 0:64] = ex[:, 0:64] * (h0 + delta[:, 0:64])
        msg_ref[:, 64:128] = ex[:, 64:128] * (h1 + delta[:, 64:128])

    full = lambda s: pl.BlockSpec(s, lambda i: (0, 0))
    return pl.pallas_call(
        body,
        grid=(grid,),
        in_specs=[
            pl.BlockSpec((blk, 128), lambda i: (i, 0)),
            pl.BlockSpec((blk, 128), lambda i: (i, 0)),
            full((128, 64)), full((8, 64)),
            full((1, 64)), full((1, 64)), full((1, 64)),
            full((64, 128)), full((1, 128)),
            full((128, 64)), full((1, 64)), full((1, 64)), full((1, 64)),
            full((64, 128)), full((1, 128)), full((64, 64)),
        ],
        out_specs=[
            pl.BlockSpec((blk, 128), lambda i: (i, 0)),
            pl.BlockSpec((blk, 128), lambda i: (i, 0)),
        ],
        out_shape=[
            jax.ShapeDtypeStruct((epad, 128), F32),
            jax.ShapeDtypeStruct((epad, 128), F32),
        ],
    )(gsrc, gdst, Wp1pad, Wp1a, bp1, gp, betap, Wp2T, bp2, Wa1T, ba1, ga,
      betaa, Wa2T, ba2, M64)


def _final(nums, dens, WoT, bo2):
    npad, d = nums[0].shape
    blk = 1280
    grid = npad // blk

    def body(n1, n2, n3, d1, d2, d3, wo, bo, out_ref):
        num = n1[...] + n2[...] + n3[...]
        den = d1[...] + d2[...] + d3[...]
        r = num / (den + 1e-16)
        out_ref[...] = jnp.maximum(_dot(r, wo[...]) + bo[...], 0.0)

    full = lambda s: pl.BlockSpec(s, lambda i: (0, 0))
    row = pl.BlockSpec((blk, d), lambda i: (i, 0))
    return pl.pallas_call(
        body,
        grid=(grid,),
        in_specs=[row, row, row, row, row, row, full((d, d)), full((1, d))],
        out_specs=row,
        out_shape=jax.ShapeDtypeStruct((npad, d), F32),
    )(*nums, *dens, WoT, bo2)


def _sc_gather(src_ids, dst_ids, src_tab, dst_tab):
    epad = src_ids.shape[0]
    wd = dst_tab.shape[1]
    bpw = epad // NW
    nch = bpw // CG
    mesh = plsc.VectorSubcoreMesh(core_axis_name="c", subcore_axis_name="s")

    assert nch % 2 == 1 and nch >= 3, nch
    npairs = (nch - 1) // 2
    assert epad % 1024 == 0

    @functools.partial(
        pl.kernel,
        out_type=(
            jax.ShapeDtypeStruct((epad, 128), jnp.int32),
            jax.ShapeDtypeStruct((epad, wd), F32),
        ),
        mesh=mesh,
        scratch_types=[
            [pltpu.VMEM((CG,), jnp.int32)] * 2,
            [pltpu.VMEM((CG,), jnp.int32)] * 2,
            [pltpu.VMEM((CG, 128), jnp.int32)] * 2,
            [pltpu.VMEM((CG, wd), F32)] * 2,
            [pltpu.SemaphoreType.DMA] * 2,
            [pltpu.SemaphoreType.DMA] * 2,
        ],
    )
    def k(sid_h, did_h, stab_h, dtab_h, gsrc_h, gdst_h, idx_s, idx_d, buf_s,
          buf_d, sem_g, sem_w):
        wid = lax.axis_index("s") * SC_CORES + lax.axis_index("c")
        base = wid * bpw

        def g_start(off, t):
            pltpu.sync_copy(sid_h.at[pl.ds(off, CG)], idx_s[t])
            pltpu.sync_copy(did_h.at[pl.ds(off, CG)], idx_d[t])
            pltpu.async_copy(stab_h.at[idx_s[t]], buf_s[t], sem_g[t])
            pltpu.async_copy(dtab_h.at[idx_d[t]], buf_d[t], sem_g[t])

        def g_wait(t):
            pltpu.make_async_copy(stab_h.at[idx_s[t]], buf_s[t], sem_g[t]).wait()
            pltpu.make_async_copy(dtab_h.at[idx_d[t]], buf_d[t], sem_g[t]).wait()

        def w_start(off, t):
            pltpu.async_copy(buf_s[t], gsrc_h.at[pl.ds(off, CG)], sem_w[t])
            pltpu.async_copy(buf_d[t], gdst_h.at[pl.ds(off, CG)], sem_w[t])

        def w_wait(off, t):
            pltpu.make_async_copy(buf_s[t], gsrc_h.at[pl.ds(off, CG)], sem_w[t]).wait()
            pltpu.make_async_copy(buf_d[t], gdst_h.at[pl.ds(off, CG)], sem_w[t]).wait()

        g_start(base, 0)

        def body(p, carry):
            offa = base + (2 * p) * CG
            offb = offa + CG
            offn = offb + CG
            g_wait(0)
            pl.when(p > 0)(lambda: w_wait(offa - CG, 1))
            g_start(offb, 1)
            w_start(offa, 0)
            g_wait(1)
            w_start(offb, 1)
            w_wait(offa, 0)
            g_start(offn, 0)
            return carry

        lax.fori_loop(0, npairs, body, 0)
        last = base + (nch - 1) * CG
        g_wait(0)
        w_wait(last - CG, 1)
        w_start(last, 0)
        w_wait(last, 0)

    return k(src_ids, dst_ids, src_tab, dst_tab)


def _sc_scatter(msg, ex, dst_ids, npad):
    epad, d = msg.shape
    per_tile = epad // SC_SUBCORES
    nch = per_tile // CS
    rows_per_tile = npad // SC_SUBCORES
    mesh = plsc.VectorSubcoreMesh(core_axis_name="c", subcore_axis_name="s")

    @functools.partial(
        pl.kernel,
        out_type=(
            jax.ShapeDtypeStruct((npad, d), F32),
            jax.ShapeDtypeStruct((npad, d), F32),
        ),
        mesh=mesh,
        scratch_types=[
            [pltpu.VMEM((CS, d), F32)] * 2,
            [pltpu.VMEM((CS,), jnp.int32)] * 2,
            pltpu.VMEM((64, d), F32),
            pltpu.VMEM_SHARED((npad, d), F32),
            [pltpu.SemaphoreType.DMA] * 2,
            [pltpu.SemaphoreType.DMA] * 2,
        ],
    )
    def k(msg_h, ex_h, did_h, num_h, den_h, rowbuf, idxb, zbuf, acc, sem_l,
          sem_a):
        c = lax.axis_index("c")
        s = lax.axis_index("s")

        for j in range(64):
            for t in range(d // 16):
                zbuf[j, pl.ds(t * 16, 16)] = jnp.zeros((16,), F32)

        def zero_body(i, carry):
            pltpu.sync_copy(zbuf, acc.at[pl.ds(s * rows_per_tile + i * 64, 64)])
            return carry

        lax.fori_loop(0, rows_per_tile // 64, zero_body, 0)
        plsc.subcore_barrier()

        def process(tbl_h):
            base = s * per_tile

            def l_start(off, t):
                pltpu.sync_copy(did_h.at[pl.ds(off, CS)], idxb[t])
                pltpu.async_copy(tbl_h.at[pl.ds(off, CS)], rowbuf[t], sem_l[t])

            def l_wait(off, t):
                pltpu.make_async_copy(
                    tbl_h.at[pl.ds(off, CS)], rowbuf[t], sem_l[t]).wait()

            def a_start(t):
                pltpu.async_copy(rowbuf[t], acc.at[idxb[t]], sem_a[t], add=True)

            def a_wait(t):
                pltpu.make_async_copy(
                    rowbuf[t], acc.at[idxb[t]], sem_a[t]).wait()

            l_start(base, 0)

            def body(p, carry):
                offa = base + (2 * p) * CS
                offb = offa + CS
                offn = offb + CS
                pl.when(p > 0)(lambda: a_wait(1))
                l_start(offb, 1)
                l_wait(offa, 0)
                a_start(0)
                l_wait(offb, 1)
                a_wait(0)
                pl.when(p + 1 < nch // 2)(lambda: l_start(offn, 0))
                a_start(1)
                return carry

            lax.fori_loop(0, nch // 2, body, 0)
            a_wait(1)

        pl.when(c == 0)(lambda: process(msg_h))
        pl.when(c == 1)(lambda: process(ex_h))
        plsc.subcore_barrier()

        def dump(out_h):
            pltpu.sync_copy(
                acc.at[pl.ds(s * rows_per_tile, rows_per_tile)],
                out_h.at[pl.ds(s * rows_per_tile, rows_per_tile)])

        pl.when(c == 0)(lambda: dump(num_h))
        pl.when(c == 1)(lambda: dump(den_h))

    return k(msg, ex, dst_ids)


def kernel(x, pos, edge_index, Wi, bi, Wo, bo, Wlin, Wsrc, Wdst, Wp1, bp1,
           gp, betap, Wp2, bp2, Wa1, ba1, ga, betaa, Wa2, ba2):
    n, dm = x.shape
    e = edge_index.shape[1]
    etot = e + n

    npad = ((n + 1 + 255) // 256) * 256
    unit = NW * CG
    epad = ((etot + unit - 1) // unit) * unit
    nchu = epad // unit
    c1 = ((nchu * 7 // 27) // 2) * 2 + 1
    cs = [c1, nchu - 2 * c1, c1]
    assert all(c % 2 == 1 and c >= 3 for c in cs), cs

    loop = jnp.arange(n, dtype=jnp.int32)
    pad_e = epad - etot
    src_full = jnp.concatenate([
        edge_index[0].astype(jnp.int32), loop,
        jnp.zeros((pad_e,), jnp.int32)])
    dst_full = jnp.concatenate([
        edge_index[1].astype(jnp.int32), loop,
        n + (jnp.arange(pad_e, dtype=jnp.int32) % (npad - n))])

    xp = jnp.pad(x, ((0, npad - n), (0, 0)))
    posp = jnp.pad(pos, ((0, npad - n), (0, 16 - pos.shape[1])))
    WiT = Wi.T
    WlinT = Wlin.T
    WsaT = (Wa1 @ Wsrc).T
    WdaT = (Wa1 @ Wdst).T
    Wp1T16 = jnp.pad(Wp1.T, ((0, 16 - Wp1.shape[1]), (0, 0)))
    Wp1a = jnp.pad(Wp1.T, ((0, 8 - Wp1.shape[1]), (0, 0)))
    Wp1pad = jnp.zeros((128, 64), F32).at[64:64 + Wp1.shape[1]].set(Wp1.T)
    M64 = jnp.full((64, 64), 1.0 / 64.0, F32)
    Wp2T = Wp2.T
    Wa1T = Wa1.T
    Wa2T = Wa2.T
    WoT = Wo.T
    r1 = lambda v: v.reshape(1, -1)

    srctab, dsttab = _node_proj(xp, posp, WiT, r1(bi), WlinT, WsaT, WdaT,
                                Wp1T16)
    nums, dens = [], []
    off = 0
    for k in range(3):
        sl = slice(off * unit, (off + cs[k]) * unit)
        off += cs[k]
        gsrc, gdst = _sc_gather(src_full[sl], dst_full[sl], srctab, dsttab)
        msg, ex = _edge_mlp(gsrc, gdst, Wp1pad, Wp1a, r1(bp1), r1(gp),
                            r1(betap), Wp2T, r1(bp2), Wa1T, r1(ba1), r1(ga),
                            r1(betaa), Wa2T, r1(ba2), M64)
        num, den = _sc_scatter(msg, ex, dst_full[sl], npad)
        nums.append(num)
        dens.append(den)
    out = _final(nums, dens, WoT, r1(bo))
    return out[:n]


# speedup vs baseline: 10.9107x; 1.0128x over previous
import functools

import jax
import jax.numpy as jnp
from jax import lax
from jax.experimental import pallas as pl
from jax.experimental.pallas import tpu as pltpu
from jax.experimental.pallas import tpu_sc as plsc

F32 = jnp.float32
BF16 = jnp.bfloat16

SC_CORES = 2
SC_SUBCORES = 16
NW = SC_CORES * SC_SUBCORES
CG = 128
CS = 128


def _ln(v, g, b):
    mu = jnp.mean(v, axis=-1, keepdims=True)
    var = jnp.mean((v - mu) ** 2, axis=-1, keepdims=True)
    return (v - mu) * lax.rsqrt(var + 1e-5) * g + b


def _ln_mm(v, g, b, m):
    mu = _dot(v, m)
    d = v - mu
    var = _dot(d * d, m)
    return d * lax.rsqrt(var + 1e-5) * g + b


def _dot(a, b):
    return jnp.dot(a, b, preferred_element_type=F32)


def _pack2(a, b):
    ua = (lax.bitcast_convert_type(a, jnp.uint32) + jnp.uint32(0x8000)) >> 16
    ub = (lax.bitcast_convert_type(b, jnp.uint32) + jnp.uint32(0x8000)) & jnp.uint32(0xFFFF0000)
    return lax.bitcast_convert_type(ua | ub, jnp.int32)


def _unpack2(u):
    uu = lax.bitcast_convert_type(u, jnp.uint32)
    a = lax.bitcast_convert_type(uu << 16, F32)
    b = lax.bitcast_convert_type(uu & jnp.uint32(0xFFFF0000), F32)
    return a, b


def _node_proj(xp, posp, WiT, bi2, WlinT, WsaT, WdaT, Wp1T16):
    npad, d = xp.shape
    blk = 1280
    grid = npad // blk

    def body(x_ref, pos_ref, wi, bi, wlin, wsa, wda, wp1, src_ref, dst_ref):
        x1 = jnp.maximum(_dot(x_ref[...], wi[...]) + bi[...], 0.0)
        h = _dot(x1, wlin[...])
        a1s = _dot(x1, wsa[...])
        psv = pos_ref[...]
        src_ref[:, 0:64] = _pack2(h[:, 0:64], h[:, 64:128])
        src_ref[:, 64:96] = _pack2(a1s[:, 0:32], a1s[:, 32:64])
        src_ref[:, 96:104] = _pack2(psv[:, 0:8], psv[:, 8:16])
        src_ref[:, 104:128] = jnp.zeros((blk, 24), jnp.int32)
        dst_ref[:, 0:64] = _dot(x1, wda[...])
        dst_ref[:, 64:80] = psv
        dst_ref[:, 80:128] = jnp.zeros((blk, 48), F32)

    full = lambda s: pl.BlockSpec(s, lambda i: (0, 0))
    return pl.pallas_call(
        body,
        grid=(grid,),
        in_specs=[
            pl.BlockSpec((blk, d), lambda i: (i, 0)),
            pl.BlockSpec((blk, 16), lambda i: (i, 0)),
            full((d, d)), full((1, d)), full((d, d)), full((d, 64)), full((d, 64)),
            full((16, 64)),
        ],
        out_specs=[
            pl.BlockSpec((blk, 128), lambda i: (i, 0)),
            pl.BlockSpec((blk, 128), lambda i: (i, 0)),
        ],
        out_shape=[
            jax.ShapeDtypeStruct((npad, 128), jnp.int32),
            jax.ShapeDtypeStruct((npad, 128), F32),
        ],
    )(xp, posp, WiT, bi2, WlinT, WsaT, WdaT, Wp1T16)


def _edge_mlp(gsrc, gdst, Wp1pad, Wp1a, bp1, gp, betap, Wp2T, bp2, Wa1T,
              ba1, ga, betaa, Wa2T, ba2, M64):
    epad = gsrc.shape[0]
    blk = 1024
    grid = epad // blk

    def body(gs_ref, gd_ref, wp1p, wp1a, bp1r, gpr, bpr, wp2, bp2r, wa1,
             ba1r, gar, bar, wa2, ba2r, m64, msg_ref, ex_ref):
        h0, h1 = _unpack2(gs_ref[:, 0:64])
        s0, s1 = _unpack2(gs_ref[:, 64:96])
        p0, _ = _unpack2(gs_ref[:, 96:104])
        gd = gd_ref[...]
        a1d = gd_ref[:, 0:64]
        t = _dot(gd, wp1p[...]) - _dot(p0, wp1a[...]) + bp1r[...]
        t = jnp.maximum(_ln_mm(t, gpr[...], bpr[...], m64[...]), 0.0)
        delta = _dot(t, wp2[...]) + bp2r[...]
        a1s = jnp.concatenate([s0, s1], axis=-1)
        a = a1d - a1s + _dot(delta, wa1[...]) + ba1r[...]
        a = jnp.maximum(_ln_mm(a, gar[...], bar[...], m64[...]), 0.0)
        alpha = _dot(a, wa2[...]) + ba2r[...]
        ex = jnp.exp(alpha)
        ex_ref[...] = ex
        msg_ref[:, 0:64] = ex[:, 0:64] * (h0 + delta[:, 0:64])
        msg_ref[:, 64:128] = ex[:, 64:128] * (h1 + delta[:, 64:128])

    full = lambda s: pl.BlockSpec(s, lambda i: (0, 0))
    return pl.pallas_call(
        body,
        grid=(grid,),
        in_specs=[
            pl.BlockSpec((blk, 128), lambda i: (i, 0)),
            pl.BlockSpec((blk, 128), lambda i: (i, 0)),
            full((128, 64)), full((8, 64)),
            full((1, 64)), full((1, 64)), full((1, 64)),
            full((64, 128)), full((1, 128)),
            full((128, 64)), full((1, 64)), full((1, 64)), full((1, 64)),
            full((64, 128)), full((1, 128)), full((64, 64)),
        ],
        out_specs=[
            pl.BlockSpec((blk, 128), lambda i: (i, 0)),
            pl.BlockSpec((blk, 128), lambda i: (i, 0)),
        ],
        out_shape=[
            jax.ShapeDtypeStruct((epad, 128), F32),
            jax.ShapeDtypeStruct((epad, 128), F32),
        ],
    )(gsrc, gdst, Wp1pad, Wp1a, bp1, gp, betap, Wp2T, bp2, Wa1T, ba1, ga,
      betaa, Wa2T, ba2, M64)


def _final(nums, dens, WoT, bo2):
    npad, d = nums[0].shape
    blk = 1280
    grid = npad // blk

    def body(n1, n2, n3, d1, d2, d3, wo, bo, out_ref):
        num = n1[...] + n2[...] + n3[...]
        den = d1[...] + d2[...] + d3[...]
        r = num / (den + 1e-16)
        out_ref[...] = jnp.maximum(_dot(r, wo[...]) + bo[...], 0.0)

    full = lambda s: pl.BlockSpec(s, lambda i: (0, 0))
    row = pl.BlockSpec((blk, d), lambda i: (i, 0))
    return pl.pallas_call(
        body,
        grid=(grid,),
        in_specs=[row, row, row, row, row, row, full((d, d)), full((1, d))],
        out_specs=row,
        out_shape=jax.ShapeDtypeStruct((npad, d), F32),
    )(*nums, *dens, WoT, bo2)


def _sc_gather(src_ids, dst_ids, src_tab, dst_tab):
    epad = src_ids.shape[0]
    wd = dst_tab.shape[1]
    bpw = epad // NW
    nch = bpw // CG
    mesh = plsc.VectorSubcoreMesh(core_axis_name="c", subcore_axis_name="s")

    assert nch % 2 == 1 and nch >= 3, nch
    npairs = (nch - 1) // 2
    assert epad % 1024 == 0

    @functools.partial(
        pl.kernel,
        out_type=(
            jax.ShapeDtypeStruct((epad, 128), jnp.int32),
            jax.ShapeDtypeStruct((epad, wd), F32),
        ),
        mesh=mesh,
        scratch_types=[
            [pltpu.VMEM((CG,), jnp.int32)] * 2,
            [pltpu.VMEM((CG,), jnp.int32)] * 2,
            [pltpu.VMEM((CG, 128), jnp.int32)] * 2,
            [pltpu.VMEM((CG, wd), F32)] * 2,
            [pltpu.SemaphoreType.DMA] * 2,
            [pltpu.SemaphoreType.DMA] * 2,
        ],
    )
    def k(sid_h, did_h, stab_h, dtab_h, gsrc_h, gdst_h, idx_s, idx_d, buf_s,
          buf_d, sem_g, sem_w):
        wid = lax.axis_index("s") * SC_CORES + lax.axis_index("c")
        base = wid * bpw

        def g_start(off, t):
            pltpu.sync_copy(sid_h.at[pl.ds(off, CG)], idx_s[t])
            pltpu.sync_copy(did_h.at[pl.ds(off, CG)], idx_d[t])
            pltpu.async_copy(stab_h.at[idx_s[t]], buf_s[t], sem_g[t])
            pltpu.async_copy(dtab_h.at[idx_d[t]], buf_d[t], sem_g[t])

        def g_wait(t):
            pltpu.make_async_copy(stab_h.at[idx_s[t]], buf_s[t], sem_g[t]).wait()
            pltpu.make_async_copy(dtab_h.at[idx_d[t]], buf_d[t], sem_g[t]).wait()

        def w_start(off, t):
            pltpu.async_copy(buf_s[t], gsrc_h.at[pl.ds(off, CG)], sem_w[t])
            pltpu.async_copy(buf_d[t], gdst_h.at[pl.ds(off, CG)], sem_w[t])

        def w_wait(off, t):
            pltpu.make_async_copy(buf_s[t], gsrc_h.at[pl.ds(off, CG)], sem_w[t]).wait()
            pltpu.make_async_copy(buf_d[t], gdst_h.at[pl.ds(off, CG)], sem_w[t]).wait()

        g_start(base, 0)

        def body(p, carry):
            offa = base + (2 * p) * CG
            offb = offa + CG
            offn = offb + CG
            g_wait(0)
            pl.when(p > 0)(lambda: w_wait(offa - CG, 1))
            g_start(offb, 1)
            w_start(offa, 0)
            g_wait(1)
            w_start(offb, 1)
            w_wait(offa, 0)
            g_start(offn, 0)
            return carry

        lax.fori_loop(0, npairs, body, 0)
        last = base + (nch - 1) * CG
        g_wait(0)
        w_wait(last - CG, 1)
        w_start(last, 0)
        w_wait(last, 0)

    return k(src_ids, dst_ids, src_tab, dst_tab)


def _sc_scatter(msg, ex, dst_ids, npad):
    epad, d = msg.shape
    per_tile = epad // SC_SUBCORES
    nch = per_tile // CS
    rows_per_tile = npad // SC_SUBCORES
    mesh = plsc.VectorSubcoreMesh(core_axis_name="c", subcore_axis_name="s")

    @functools.partial(
        pl.kernel,
        out_type=(
            jax.ShapeDtypeStruct((npad, d), F32),
            jax.ShapeDtypeStruct((npad, d), F32),
        ),
        mesh=mesh,
        scratch_types=[
            [pltpu.VMEM((CS, d), F32)] * 2,
            [pltpu.VMEM((CS,), jnp.int32)] * 2,
            pltpu.VMEM((64, d), F32),
            pltpu.VMEM_SHARED((npad, d), F32),
            [pltpu.SemaphoreType.DMA] * 2,
            [pltpu.SemaphoreType.DMA] * 2,
        ],
    )
    def k(msg_h, ex_h, did_h, num_h, den_h, rowbuf, idxb, zbuf, acc, sem_l,
          sem_a):
        c = lax.axis_index("c")
        s = lax.axis_index("s")

        for j in range(64):
            for t in range(d // 16):
                zbuf[j, pl.ds(t * 16, 16)] = jnp.zeros((16,), F32)

        def zero_body(i, carry):
            pltpu.sync_copy(zbuf, acc.at[pl.ds(s * rows_per_tile + i * 64, 64)])
            return carry

        lax.fori_loop(0, rows_per_tile // 64, zero_body, 0)
        plsc.subcore_barrier()

        def process(tbl_h):
            base = s * per_tile

            def l_start(off, t):
                pltpu.sync_copy(did_h.at[pl.ds(off, CS)], idxb[t])
                pltpu.async_copy(tbl_h.at[pl.ds(off, CS)], rowbuf[t], sem_l[t])

            def l_wait(off, t):
                pltpu.make_async_copy(
                    tbl_h.at[pl.ds(off, CS)], rowbuf[t], sem_l[t]).wait()

            def a_start(t):
                pltpu.async_copy(rowbuf[t], acc.at[idxb[t]], sem_a[t], add=True)

            def a_wait(t):
                pltpu.make_async_copy(
                    rowbuf[t], acc.at[idxb[t]], sem_a[t]).wait()

            l_start(base, 0)

            def body(p, carry):
                offa = base + (2 * p) * CS
                offb = offa + CS
                offn = offb + CS
                pl.when(p > 0)(lambda: a_wait(1))
                l_start(offb, 1)
                l_wait(offa, 0)
                a_start(0)
                l_wait(offb, 1)
                a_start(1)
                a_wait(0)
                pl.when(p + 1 < nch // 2)(lambda: l_start(offn, 0))
                return carry

            lax.fori_loop(0, nch // 2, body, 0)
            a_wait(1)

        pl.when(c == 0)(lambda: process(msg_h))
        pl.when(c == 1)(lambda: process(ex_h))
        plsc.subcore_barrier()

        def dump(out_h):
            pltpu.sync_copy(
                acc.at[pl.ds(s * rows_per_tile, rows_per_tile)],
                out_h.at[pl.ds(s * rows_per_tile, rows_per_tile)])

        pl.when(c == 0)(lambda: dump(num_h))
        pl.when(c == 1)(lambda: dump(den_h))

    return k(msg, ex, dst_ids)


def kernel(x, pos, edge_index, Wi, bi, Wo, bo, Wlin, Wsrc, Wdst, Wp1, bp1,
           gp, betap, Wp2, bp2, Wa1, ba1, ga, betaa, Wa2, ba2):
    n, dm = x.shape
    e = edge_index.shape[1]
    etot = e + n

    npad = ((n + 1 + 255) // 256) * 256
    unit = NW * CG
    epad = ((etot + unit - 1) // unit) * unit
    nchu = epad // unit
    c1 = ((nchu * 7 // 27) // 2) * 2 + 1
    cs = [c1, nchu - 2 * c1, c1]
    assert all(c % 2 == 1 and c >= 3 for c in cs), cs

    loop = jnp.arange(n, dtype=jnp.int32)
    pad_e = epad - etot
    src_full = jnp.concatenate([
        edge_index[0].astype(jnp.int32), loop,
        jnp.zeros((pad_e,), jnp.int32)])
    dst_full = jnp.concatenate([
        edge_index[1].astype(jnp.int32), loop,
        n + (jnp.arange(pad_e, dtype=jnp.int32) % (npad - n))])

    xp = jnp.pad(x, ((0, npad - n), (0, 0)))
    posp = jnp.pad(pos, ((0, npad - n), (0, 16 - pos.shape[1])))
    WiT = Wi.T
    WlinT = Wlin.T
    WsaT = (Wa1 @ Wsrc).T
    WdaT = (Wa1 @ Wdst).T
    Wp1T16 = jnp.pad(Wp1.T, ((0, 16 - Wp1.shape[1]), (0, 0)))
    Wp1a = jnp.pad(Wp1.T, ((0, 8 - Wp1.shape[1]), (0, 0)))
    Wp1pad = jnp.zeros((128, 64), F32).at[64:64 + Wp1.shape[1]].set(Wp1.T)
    M64 = jnp.full((64, 64), 1.0 / 64.0, F32)
    Wp2T = Wp2.T
    Wa1T = Wa1.T
    Wa2T = Wa2.T
    WoT = Wo.T
    r1 = lambda v: v.reshape(1, -1)

    srctab, dsttab = _node_proj(xp, posp, WiT, r1(bi), WlinT, WsaT, WdaT,
                                Wp1T16)
    nums, dens = [], []
    off = 0
    for k in range(3):
        sl = slice(off * unit, (off + cs[k]) * unit)
        off += cs[k]
        gsrc, gdst = _sc_gather(src_full[sl], dst_full[sl], srctab, dsttab)
        msg, ex = _edge_mlp(gsrc, gdst, Wp1pad, Wp1a, r1(bp1), r1(gp),
                            r1(betap), Wp2T, r1(bp2), Wa1T, r1(ba1), r1(ga),
                            r1(betaa), Wa2T, r1(ba2), M64)
        num, den = _sc_scatter(msg, ex, dst_full[sl], npad)
        nums.append(num)
        dens.append(den)
    out = _final(nums, dens, WoT, r1(bo))
    return out[:n]


# speedup vs baseline: 11.0124x; 1.0093x over previous
import functools

import jax
import jax.numpy as jnp
from jax import lax
from jax.experimental import pallas as pl
from jax.experimental.pallas import tpu as pltpu
from jax.experimental.pallas import tpu_sc as plsc

F32 = jnp.float32
BF16 = jnp.bfloat16

SC_CORES = 2
SC_SUBCORES = 16
NW = SC_CORES * SC_SUBCORES
CG = 128
CS = 128


def _ln(v, g, b):
    mu = jnp.mean(v, axis=-1, keepdims=True)
    var = jnp.mean((v - mu) ** 2, axis=-1, keepdims=True)
    return (v - mu) * lax.rsqrt(var + 1e-5) * g + b


def _ln_mm(v, g, b, m):
    mu = _dot(v, m)
    d = v - mu
    var = _dot(d * d, m)
    return d * lax.rsqrt(var + 1e-5) * g + b


def _dot(a, b):
    return jnp.dot(a, b, preferred_element_type=F32)


def _pack2(a, b):
    ua = (lax.bitcast_convert_type(a, jnp.uint32) + jnp.uint32(0x8000)) >> 16
    ub = (lax.bitcast_convert_type(b, jnp.uint32) + jnp.uint32(0x8000)) & jnp.uint32(0xFFFF0000)
    return lax.bitcast_convert_type(ua | ub, jnp.int32)


def _unpack2(u):
    uu = lax.bitcast_convert_type(u, jnp.uint32)
    a = lax.bitcast_convert_type(uu << 16, F32)
    b = lax.bitcast_convert_type(uu & jnp.uint32(0xFFFF0000), F32)
    return a, b


def _node_proj(xp, posp, WiT, bi2, WlinT, WsaT, WdaT, Wp1T16):
    npad, d = xp.shape
    blk = 1280
    grid = npad // blk

    def body(x_ref, pos_ref, wi, bi, wlin, wsa, wda, wp1, src_ref, dst_ref):
        x1 = jnp.maximum(_dot(x_ref[...], wi[...]) + bi[...], 0.0)
        h = _dot(x1, wlin[...])
        a1s = _dot(x1, wsa[...])
        psv = pos_ref[...]
        src_ref[:, 0:64] = _pack2(h[:, 0:64], h[:, 64:128])
        src_ref[:, 64:96] = _pack2(a1s[:, 0:32], a1s[:, 32:64])
        src_ref[:, 96:104] = _pack2(psv[:, 0:8], psv[:, 8:16])
        src_ref[:, 104:128] = jnp.zeros((blk, 24), jnp.int32)
        dst_ref[:, 0:64] = _dot(x1, wda[...])
        dst_ref[:, 64:80] = psv
        dst_ref[:, 80:128] = jnp.zeros((blk, 48), F32)

    full = lambda s: pl.BlockSpec(s, lambda i: (0, 0))
    return pl.pallas_call(
        body,
        grid=(grid,),
        in_specs=[
            pl.BlockSpec((blk, d), lambda i: (i, 0)),
            pl.BlockSpec((blk, 16), lambda i: (i, 0)),
            full((d, d)), full((1, d)), full((d, d)), full((d, 64)), full((d, 64)),
            full((16, 64)),
        ],
        out_specs=[
            pl.BlockSpec((blk, 128), lambda i: (i, 0)),
            pl.BlockSpec((blk, 128), lambda i: (i, 0)),
        ],
        out_shape=[
            jax.ShapeDtypeStruct((npad, 128), jnp.int32),
            jax.ShapeDtypeStruct((npad, 128), F32),
        ],
    )(xp, posp, WiT, bi2, WlinT, WsaT, WdaT, Wp1T16)


def _edge_mlp(gsrc, gdst, Wp1pad, Wp1a, bp1, gp, betap, Wp2T, bp2, Wa1T,
              ba1, ga, betaa, Wa2T, ba2, M64):
    epad = gsrc.shape[0]
    blk = 1024
    grid = epad // blk

    def body(gs_ref, gd_ref, wp1p, wp1a, bp1r, gpr, bpr, wp2, bp2r, wa1,
             ba1r, gar, bar, wa2, ba2r, m64, msg_ref, ex_ref):
        h0, h1 = _unpack2(gs_ref[:, 0:64])
        s0, s1 = _unpack2(gs_ref[:, 64:96])
        p0, _ = _unpack2(gs_ref[:, 96:104])
        gd = gd_ref[...]
        a1d = gd_ref[:, 0:64]
        t = _dot(gd, wp1p[...]) - _dot(p0, wp1a[...]) + bp1r[...]
        t = jnp.maximum(_ln_mm(t, gpr[...], bpr[...], m64[...]), 0.0)
        delta = _dot(t, wp2[...]) + bp2r[...]
        a1s = jnp.concatenate([s0, s1], axis=-1)
        a = a1d - a1s + _dot(delta, wa1[...]) + ba1r[...]
        a = jnp.maximum(_ln_mm(a, gar[...], bar[...], m64[...]), 0.0)
        alpha = _dot(a, wa2[...]) + ba2r[...]
        ex = jnp.exp(alpha)
        ex_ref[...] = ex
        msg_ref[:, 0:64] = ex[:, 0:64] * (h0 + delta[:, 0:64])
        msg_ref[:, 64:128] = ex[:, 64:128] * (h1 + delta[:, 64:128])

    full = lambda s: pl.BlockSpec(s, lambda i: (0, 0))
    return pl.pallas_call(
        body,
        grid=(grid,),
        in_specs=[
            pl.BlockSpec((blk, 128), lambda i: (i, 0)),
            pl.BlockSpec((blk, 128), lambda i: (i, 0)),
            full((128, 64)), full((8, 64)),
            full((1, 64)), full((1, 64)), full((1, 64)),
            full((64, 128)), full((1, 128)),
            full((128, 64)), full((1, 64)), full((1, 64)), full((1, 64)),
            full((64, 128)), full((1, 128)), full((64, 64)),
        ],
        out_specs=[
            pl.BlockSpec((blk, 128), lambda i: (i, 0)),
            pl.BlockSpec((blk, 128), lambda i: (i, 0)),
        ],
        out_shape=[
            jax.ShapeDtypeStruct((epad, 128), F32),
            jax.ShapeDtypeStruct((epad, 128), F32),
        ],
    )(gsrc, gdst, Wp1pad, Wp1a, bp1, gp, betap, Wp2T, bp2, Wa1T, ba1, ga,
      betaa, Wa2T, ba2, M64)


def _final(nums, dens, WoT, bo2):
    npad, d = nums[0].shape
    blk = 1280
    grid = npad // blk

    k = len(nums)

    def body(*refs):
        nrefs = refs[:k]
        drefs = refs[k:2 * k]
        wo, bo, out_ref = refs[2 * k], refs[2 * k + 1], refs[2 * k + 2]
        num = sum(r[...] for r in nrefs[1:]) + nrefs[0][...]
        den = sum(r[...] for r in drefs[1:]) + drefs[0][...]
        r = num / (den + 1e-16)
        out_ref[...] = jnp.maximum(_dot(r, wo[...]) + bo[...], 0.0)

    full = lambda s: pl.BlockSpec(s, lambda i: (0, 0))
    row = pl.BlockSpec((blk, d), lambda i: (i, 0))
    return pl.pallas_call(
        body,
        grid=(grid,),
        in_specs=[row] * (2 * k) + [full((d, d)), full((1, d))],
        out_specs=row,
        out_shape=jax.ShapeDtypeStruct((npad, d), F32),
    )(*nums, *dens, WoT, bo2)


def _sc_gather(src_ids, dst_ids, src_tab, dst_tab):
    epad = src_ids.shape[0]
    wd = dst_tab.shape[1]
    bpw = epad // NW
    nch = bpw // CG
    mesh = plsc.VectorSubcoreMesh(core_axis_name="c", subcore_axis_name="s")

    assert nch % 2 == 1 and nch >= 3, nch
    npairs = (nch - 1) // 2
    assert epad % 1024 == 0

    @functools.partial(
        pl.kernel,
        out_type=(
            jax.ShapeDtypeStruct((epad, 128), jnp.int32),
            jax.ShapeDtypeStruct((epad, wd), F32),
        ),
        mesh=mesh,
        scratch_types=[
            [pltpu.VMEM((CG,), jnp.int32)] * 2,
            [pltpu.VMEM((CG,), jnp.int32)] * 2,
            [pltpu.VMEM((CG, 128), jnp.int32)] * 2,
            [pltpu.VMEM((CG, wd), F32)] * 2,
            [pltpu.SemaphoreType.DMA] * 2,
            [pltpu.SemaphoreType.DMA] * 2,
        ],
    )
    def k(sid_h, did_h, stab_h, dtab_h, gsrc_h, gdst_h, idx_s, idx_d, buf_s,
          buf_d, sem_g, sem_w):
        wid = lax.axis_index("s") * SC_CORES + lax.axis_index("c")
        base = wid * bpw

        def g_start(off, t):
            pltpu.sync_copy(sid_h.at[pl.ds(off, CG)], idx_s[t])
            pltpu.sync_copy(did_h.at[pl.ds(off, CG)], idx_d[t])
            pltpu.async_copy(stab_h.at[idx_s[t]], buf_s[t], sem_g[t])
            pltpu.async_copy(dtab_h.at[idx_d[t]], buf_d[t], sem_g[t])

        def g_wait(t):
            pltpu.make_async_copy(stab_h.at[idx_s[t]], buf_s[t], sem_g[t]).wait()
            pltpu.make_async_copy(dtab_h.at[idx_d[t]], buf_d[t], sem_g[t]).wait()

        def w_start(off, t):
            pltpu.async_copy(buf_s[t], gsrc_h.at[pl.ds(off, CG)], sem_w[t])
            pltpu.async_copy(buf_d[t], gdst_h.at[pl.ds(off, CG)], sem_w[t])

        def w_wait(off, t):
            pltpu.make_async_copy(buf_s[t], gsrc_h.at[pl.ds(off, CG)], sem_w[t]).wait()
            pltpu.make_async_copy(buf_d[t], gdst_h.at[pl.ds(off, CG)], sem_w[t]).wait()

        g_start(base, 0)

        def body(p, carry):
            offa = base + (2 * p) * CG
            offb = offa + CG
            offn = offb + CG
            g_wait(0)
            pl.when(p > 0)(lambda: w_wait(offa - CG, 1))
            g_start(offb, 1)
            w_start(offa, 0)
            g_wait(1)
            w_start(offb, 1)
            w_wait(offa, 0)
            g_start(offn, 0)
            return carry

        lax.fori_loop(0, npairs, body, 0)
        last = base + (nch - 1) * CG
        g_wait(0)
        w_wait(last - CG, 1)
        w_start(last, 0)
        w_wait(last, 0)

    return k(src_ids, dst_ids, src_tab, dst_tab)


def _sc_scatter(msg, ex, dst_ids, npad):
    epad, d = msg.shape
    per_tile = epad // SC_SUBCORES
    nch = per_tile // CS
    rows_per_tile = npad // SC_SUBCORES
    mesh = plsc.VectorSubcoreMesh(core_axis_name="c", subcore_axis_name="s")

    @functools.partial(
        pl.kernel,
        out_type=(
            jax.ShapeDtypeStruct((npad, d), F32),
            jax.ShapeDtypeStruct((npad, d), F32),
        ),
        mesh=mesh,
        scratch_types=[
            [pltpu.VMEM((CS, d), F32)] * 2,
            [pltpu.VMEM((CS,), jnp.int32)] * 2,
            pltpu.VMEM((64, d), F32),
            pltpu.VMEM_SHARED((npad, d), F32),
            [pltpu.SemaphoreType.DMA] * 2,
            [pltpu.SemaphoreType.DMA] * 2,
        ],
    )
    def k(msg_h, ex_h, did_h, num_h, den_h, rowbuf, idxb, zbuf, acc, sem_l,
          sem_a):
        c = lax.axis_index("c")
        s = lax.axis_index("s")

        for j in range(64):
            for t in range(d // 16):
                zbuf[j, pl.ds(t * 16, 16)] = jnp.zeros((16,), F32)

        def zero_body(i, carry):
            pltpu.sync_copy(zbuf, acc.at[pl.ds(s * rows_per_tile + i * 64, 64)])
            return carry

        lax.fori_loop(0, rows_per_tile // 64, zero_body, 0)
        plsc.subcore_barrier()

        def process(tbl_h):
            base = s * per_tile

            def l_start(off, t):
                pltpu.sync_copy(did_h.at[pl.ds(off, CS)], idxb[t])
                pltpu.async_copy(tbl_h.at[pl.ds(off, CS)], rowbuf[t], sem_l[t])

            def l_wait(off, t):
                pltpu.make_async_copy(
                    tbl_h.at[pl.ds(off, CS)], rowbuf[t], sem_l[t]).wait()

            def a_start(t):
                pltpu.async_copy(rowbuf[t], acc.at[idxb[t]], sem_a[t], add=True)

            def a_wait(t):
                pltpu.make_async_copy(
                    rowbuf[t], acc.at[idxb[t]], sem_a[t]).wait()

            l_start(base, 0)

            def body(p, carry):
                offa = base + (2 * p) * CS
                offb = offa + CS
                offn = offb + CS
                pl.when(p > 0)(lambda: a_wait(1))
                l_start(offb, 1)
                l_wait(offa, 0)
                a_start(0)
                l_wait(offb, 1)
                a_start(1)
                a_wait(0)
                pl.when(p + 1 < nch // 2)(lambda: l_start(offn, 0))
                return carry

            lax.fori_loop(0, nch // 2, body, 0)
            a_wait(1)

        pl.when(c == 0)(lambda: process(msg_h))
        pl.when(c == 1)(lambda: process(ex_h))
        plsc.subcore_barrier()

        def dump(out_h):
            pltpu.sync_copy(
                acc.at[pl.ds(s * rows_per_tile, rows_per_tile)],
                out_h.at[pl.ds(s * rows_per_tile, rows_per_tile)])

        pl.when(c == 0)(lambda: dump(num_h))
        pl.when(c == 1)(lambda: dump(den_h))

    return k(msg, ex, dst_ids)


def kernel(x, pos, edge_index, Wi, bi, Wo, bo, Wlin, Wsrc, Wdst, Wp1, bp1,
           gp, betap, Wp2, bp2, Wa1, ba1, ga, betaa, Wa2, ba2):
    n, dm = x.shape
    e = edge_index.shape[1]
    etot = e + n

    npad = ((n + 1 + 255) // 256) * 256
    unit = NW * CG
    epad = ((etot + unit - 1) // unit) * unit
    nchu = epad // unit
    c1 = ((nchu // 9) // 2) * 2 + 1
    cm = (nchu - 2 * c1) // 3
    cs = [c1, cm, cm, nchu - 2 * c1 - 2 * cm, c1]
    assert sum(cs) == nchu and all(c % 2 == 1 and c >= 3 for c in cs), cs

    loop = jnp.arange(n, dtype=jnp.int32)
    pad_e = epad - etot
    src_full = jnp.concatenate([
        edge_index[0].astype(jnp.int32), loop,
        jnp.zeros((pad_e,), jnp.int32)])
    dst_full = jnp.concatenate([
        edge_index[1].astype(jnp.int32), loop,
        n + (jnp.arange(pad_e, dtype=jnp.int32) % (npad - n))])

    xp = jnp.pad(x, ((0, npad - n), (0, 0)))
    posp = jnp.pad(pos, ((0, npad - n), (0, 16 - pos.shape[1])))
    WiT = Wi.T
    WlinT = Wlin.T
    WsaT = (Wa1 @ Wsrc).T
    WdaT = (Wa1 @ Wdst).T
    Wp1T16 = jnp.pad(Wp1.T, ((0, 16 - Wp1.shape[1]), (0, 0)))
    Wp1a = jnp.pad(Wp1.T, ((0, 8 - Wp1.shape[1]), (0, 0)))
    Wp1pad = jnp.zeros((128, 64), F32).at[64:64 + Wp1.shape[1]].set(Wp1.T)
    M64 = jnp.full((64, 64), 1.0 / 64.0, F32)
    Wp2T = Wp2.T
    Wa1T = Wa1.T
    Wa2T = Wa2.T
    WoT = Wo.T
    r1 = lambda v: v.reshape(1, -1)

    srctab, dsttab = _node_proj(xp, posp, WiT, r1(bi), WlinT, WsaT, WdaT,
                                Wp1T16)
    nums, dens = [], []
    off = 0
    for ck in cs:
        sl = slice(off * unit, (off + ck) * unit)
        off += ck
        gsrc, gdst = _sc_gather(src_full[sl], dst_full[sl], srctab, dsttab)
        msg, ex = _edge_mlp(gsrc, gdst, Wp1pad, Wp1a, r1(bp1), r1(gp),
                            r1(betap), Wp2T, r1(bp2), Wa1T, r1(ba1), r1(ga),
                            r1(betaa), Wa2T, r1(ba2), M64)
        num, den = _sc_scatter(msg, ex, dst_full[sl], npad)
        nums.append(num)
        dens.append(den)
    out = _final(nums, dens, WoT, r1(bo))
    return out[:n]


# speedup vs baseline: 11.0428x; 1.0028x over previous
import functools

import jax
import jax.numpy as jnp
from jax import lax
from jax.experimental import pallas as pl
from jax.experimental.pallas import tpu as pltpu
from jax.experimental.pallas import tpu_sc as plsc

F32 = jnp.float32

SC_CORES = 2
SC_SUBCORES = 16
NW = SC_CORES * SC_SUBCORES
CG = 128
CS = 128


def _ln_mm(v, g, b, m):
    mu = _dot(v, m)
    d = v - mu
    var = _dot(d * d, m)
    return d * lax.rsqrt(var + 1e-5) * g + b


def _dot(a, b):
    return jnp.dot(a, b, preferred_element_type=F32)


def _pack2(a, b):
    ua = (lax.bitcast_convert_type(a, jnp.uint32) + jnp.uint32(0x8000)) >> 16
    ub = (lax.bitcast_convert_type(b, jnp.uint32) + jnp.uint32(0x8000)) & jnp.uint32(0xFFFF0000)
    return lax.bitcast_convert_type(ua | ub, jnp.int32)


def _unpack2(u):
    uu = lax.bitcast_convert_type(u, jnp.uint32)
    a = lax.bitcast_convert_type(uu << 16, F32)
    b = lax.bitcast_convert_type(uu & jnp.uint32(0xFFFF0000), F32)
    return a, b


def _node_proj(xp, posp, WiT, bi2, WlinT, WsaT, WdaT, Wp1T16):
    npad, d = xp.shape
    blk = 1280
    grid = npad // blk

    def body(x_ref, pos_ref, wi, bi, wlin, wsa, wda, wp1, src_ref, dst_ref):
        x1 = jnp.maximum(_dot(x_ref[...], wi[...]) + bi[...], 0.0)
        h = _dot(x1, wlin[...])
        a1s = _dot(x1, wsa[...])
        psv = pos_ref[...]
        src_ref[:, 0:64] = _pack2(h[:, 0:64], h[:, 64:128])
        src_ref[:, 64:96] = _pack2(a1s[:, 0:32], a1s[:, 32:64])
        src_ref[:, 96:104] = _pack2(psv[:, 0:8], psv[:, 8:16])
        src_ref[:, 104:128] = jnp.zeros((blk, 24), jnp.int32)
        dst_ref[:, 0:64] = _dot(x1, wda[...])
        dst_ref[:, 64:80] = psv
        dst_ref[:, 80:128] = jnp.zeros((blk, 48), F32)

    full = lambda s: pl.BlockSpec(s, lambda i: (0, 0))
    return pl.pallas_call(
        body,
        grid=(grid,),
        in_specs=[
            pl.BlockSpec((blk, d), lambda i: (i, 0)),
            pl.BlockSpec((blk, 16), lambda i: (i, 0)),
            full((d, d)), full((1, d)), full((d, d)), full((d, 64)), full((d, 64)),
            full((16, 64)),
        ],
        out_specs=[
            pl.BlockSpec((blk, 128), lambda i: (i, 0)),
            pl.BlockSpec((blk, 128), lambda i: (i, 0)),
        ],
        out_shape=[
            jax.ShapeDtypeStruct((npad, 128), jnp.int32),
            jax.ShapeDtypeStruct((npad, 128), F32),
        ],
    )(xp, posp, WiT, bi2, WlinT, WsaT, WdaT, Wp1T16)


def _edge_mlp(gsrc, gdst, Wp1pad, Wp1a, bp1, gp, betap, Wp2T, bp2, Wa1T,
              ba1, ga, betaa, Wa2T, ba2, M64):
    epad = gsrc.shape[0]
    blk = 1024
    grid = epad // blk

    def body(gs_ref, gd_ref, wp1p, wp1a, bp1r, gpr, bpr, wp2, bp2r, wa1,
             ba1r, gar, bar, wa2, ba2r, m64, msg_ref, ex_ref):
        h0, h1 = _unpack2(gs_ref[:, 0:64])
        s0, s1 = _unpack2(gs_ref[:, 64:96])
        p0, _ = _unpack2(gs_ref[:, 96:104])
        gd = gd_ref[...]
        a1d = gd_ref[:, 0:64]
        t = _dot(gd, wp1p[...]) - _dot(p0, wp1a[...]) + bp1r[...]
        t = jnp.maximum(_ln_mm(t, gpr[...], bpr[...], m64[...]), 0.0)
        delta = _dot(t, wp2[...]) + bp2r[...]
        a1s = jnp.concatenate([s0, s1], axis=-1)
        a = a1d - a1s + _dot(delta, wa1[...]) + ba1r[...]
        a = jnp.maximum(_ln_mm(a, gar[...], bar[...], m64[...]), 0.0)
        alpha = _dot(a, wa2[...]) + ba2r[...]
        ex = jnp.exp(alpha)
        ex_ref[...] = ex
        msg_ref[:, 0:64] = ex[:, 0:64] * (h0 + delta[:, 0:64])
        msg_ref[:, 64:128] = ex[:, 64:128] * (h1 + delta[:, 64:128])

    full = lambda s: pl.BlockSpec(s, lambda i: (0, 0))
    return pl.pallas_call(
        body,
        grid=(grid,),
        in_specs=[
            pl.BlockSpec((blk, 128), lambda i: (i, 0)),
            pl.BlockSpec((blk, 128), lambda i: (i, 0)),
            full((128, 64)), full((8, 64)),
            full((1, 64)), full((1, 64)), full((1, 64)),
            full((64, 128)), full((1, 128)),
            full((128, 64)), full((1, 64)), full((1, 64)), full((1, 64)),
            full((64, 128)), full((1, 128)), full((64, 64)),
        ],
        out_specs=[
            pl.BlockSpec((blk, 128), lambda i: (i, 0)),
            pl.BlockSpec((blk, 128), lambda i: (i, 0)),
        ],
        out_shape=[
            jax.ShapeDtypeStruct((epad, 128), F32),
            jax.ShapeDtypeStruct((epad, 128), F32),
        ],
    )(gsrc, gdst, Wp1pad, Wp1a, bp1, gp, betap, Wp2T, bp2, Wa1T, ba1, ga,
      betaa, Wa2T, ba2, M64)


def _final(nums, dens, WoT, bo2):
    npad, d = nums[0].shape
    blk = 1280
    grid = npad // blk

    k = len(nums)

    def body(*refs):
        nrefs = refs[:k]
        drefs = refs[k:2 * k]
        wo, bo, out_ref = refs[2 * k], refs[2 * k + 1], refs[2 * k + 2]
        num = sum(r[...] for r in nrefs[1:]) + nrefs[0][...]
        den = sum(r[...] for r in drefs[1:]) + drefs[0][...]
        r = num / (den + 1e-16)
        out_ref[...] = jnp.maximum(_dot(r, wo[...]) + bo[...], 0.0)

    full = lambda s: pl.BlockSpec(s, lambda i: (0, 0))
    row = pl.BlockSpec((blk, d), lambda i: (i, 0))
    return pl.pallas_call(
        body,
        grid=(grid,),
        in_specs=[row] * (2 * k) + [full((d, d)), full((1, d))],
        out_specs=row,
        out_shape=jax.ShapeDtypeStruct((npad, d), F32),
    )(*nums, *dens, WoT, bo2)


def _sc_gather(src_ids, dst_ids, src_tab, dst_tab):
    epad = src_ids.shape[0]
    wd = dst_tab.shape[1]
    bpw = epad // NW
    nch = bpw // CG
    mesh = plsc.VectorSubcoreMesh(core_axis_name="c", subcore_axis_name="s")

    assert nch % 2 == 1 and nch >= 3, nch
    npairs = (nch - 1) // 2
    assert epad % 1024 == 0

    @functools.partial(
        pl.kernel,
        out_type=(
            jax.ShapeDtypeStruct((epad, 128), jnp.int32),
            jax.ShapeDtypeStruct((epad, wd), F32),
        ),
        mesh=mesh,
        scratch_types=[
            [pltpu.VMEM((CG,), jnp.int32)] * 2,
            [pltpu.VMEM((CG,), jnp.int32)] * 2,
            [pltpu.VMEM((CG, 128), jnp.int32)] * 2,
            [pltpu.VMEM((CG, wd), F32)] * 2,
            [pltpu.SemaphoreType.DMA] * 2,
            [pltpu.SemaphoreType.DMA] * 2,
        ],
    )
    def k(sid_h, did_h, stab_h, dtab_h, gsrc_h, gdst_h, idx_s, idx_d, buf_s,
          buf_d, sem_g, sem_w):
        wid = lax.axis_index("s") * SC_CORES + lax.axis_index("c")
        base = wid * bpw

        def g_start(off, t):
            pltpu.sync_copy(sid_h.at[pl.ds(off, CG)], idx_s[t])
            pltpu.sync_copy(did_h.at[pl.ds(off, CG)], idx_d[t])
            pltpu.async_copy(stab_h.at[idx_s[t]], buf_s[t], sem_g[t])
            pltpu.async_copy(dtab_h.at[idx_d[t]], buf_d[t], sem_g[t])

        def g_wait(t):
            pltpu.make_async_copy(stab_h.at[idx_s[t]], buf_s[t], sem_g[t]).wait()
            pltpu.make_async_copy(dtab_h.at[idx_d[t]], buf_d[t], sem_g[t]).wait()

        def w_start(off, t):
            pltpu.async_copy(buf_s[t], gsrc_h.at[pl.ds(off, CG)], sem_w[t])
            pltpu.async_copy(buf_d[t], gdst_h.at[pl.ds(off, CG)], sem_w[t])

        def w_wait(off, t):
            pltpu.make_async_copy(buf_s[t], gsrc_h.at[pl.ds(off, CG)], sem_w[t]).wait()
            pltpu.make_async_copy(buf_d[t], gdst_h.at[pl.ds(off, CG)], sem_w[t]).wait()

        g_start(base, 0)

        def body(p, carry):
            offa = base + (2 * p) * CG
            offb = offa + CG
            offn = offb + CG
            g_wait(0)
            pl.when(p > 0)(lambda: w_wait(offa - CG, 1))
            g_start(offb, 1)
            w_start(offa, 0)
            g_wait(1)
            w_start(offb, 1)
            w_wait(offa, 0)
            g_start(offn, 0)
            return carry

        lax.fori_loop(0, npairs, body, 0)
        last = base + (nch - 1) * CG
        g_wait(0)
        w_wait(last - CG, 1)
        w_start(last, 0)
        w_wait(last, 0)

    return k(src_ids, dst_ids, src_tab, dst_tab)


def _sc_scatter(msg, ex, dst_ids, npad):
    epad, d = msg.shape
    per_tile = epad // SC_SUBCORES
    nch = per_tile // CS
    rows_per_tile = npad // SC_SUBCORES
    mesh = plsc.VectorSubcoreMesh(core_axis_name="c", subcore_axis_name="s")

    @functools.partial(
        pl.kernel,
        out_type=(
            jax.ShapeDtypeStruct((npad, d), F32),
            jax.ShapeDtypeStruct((npad, d), F32),
        ),
        mesh=mesh,
        scratch_types=[
            [pltpu.VMEM((CS, d), F32)] * 2,
            [pltpu.VMEM((CS,), jnp.int32)] * 2,
            pltpu.VMEM((64, d), F32),
            pltpu.VMEM_SHARED((npad, d), F32),
            [pltpu.SemaphoreType.DMA] * 2,
            [pltpu.SemaphoreType.DMA] * 2,
        ],
    )
    def k(msg_h, ex_h, did_h, num_h, den_h, rowbuf, idxb, zbuf, acc, sem_l,
          sem_a):
        c = lax.axis_index("c")
        s = lax.axis_index("s")

        for j in range(64):
            for t in range(d // 16):
                zbuf[j, pl.ds(t * 16, 16)] = jnp.zeros((16,), F32)

        def zero_body(i, carry):
            pltpu.sync_copy(zbuf, acc.at[pl.ds(s * rows_per_tile + i * 64, 64)])
            return carry

        lax.fori_loop(0, rows_per_tile // 64, zero_body, 0)
        plsc.subcore_barrier()

        def process(tbl_h):
            base = s * per_tile

            def l_start(off, t):
                pltpu.sync_copy(did_h.at[pl.ds(off, CS)], idxb[t])
                pltpu.async_copy(tbl_h.at[pl.ds(off, CS)], rowbuf[t], sem_l[t])

            def l_wait(off, t):
                pltpu.make_async_copy(
                    tbl_h.at[pl.ds(off, CS)], rowbuf[t], sem_l[t]).wait()

            def a_start(t):
                pltpu.async_copy(rowbuf[t], acc.at[idxb[t]], sem_a[t], add=True)

            def a_wait(t):
                pltpu.make_async_copy(
                    rowbuf[t], acc.at[idxb[t]], sem_a[t]).wait()

            l_start(base, 0)

            def body(p, carry):
                offa = base + (2 * p) * CS
                offb = offa + CS
                offn = offb + CS
                pl.when(p > 0)(lambda: a_wait(1))
                l_start(offb, 1)
                l_wait(offa, 0)
                a_start(0)
                l_wait(offb, 1)
                a_start(1)
                a_wait(0)
                pl.when(p + 1 < nch // 2)(lambda: l_start(offn, 0))
                return carry

            lax.fori_loop(0, nch // 2, body, 0)
            a_wait(1)

        pl.when(c == 0)(lambda: process(msg_h))
        pl.when(c == 1)(lambda: process(ex_h))
        plsc.subcore_barrier()

        def dump(out_h):
            pltpu.sync_copy(
                acc.at[pl.ds(s * rows_per_tile, rows_per_tile)],
                out_h.at[pl.ds(s * rows_per_tile, rows_per_tile)])

        pl.when(c == 0)(lambda: dump(num_h))
        pl.when(c == 1)(lambda: dump(den_h))

    return k(msg, ex, dst_ids)


def kernel(x, pos, edge_index, Wi, bi, Wo, bo, Wlin, Wsrc, Wdst, Wp1, bp1,
           gp, betap, Wp2, bp2, Wa1, ba1, ga, betaa, Wa2, ba2):
    n, dm = x.shape
    e = edge_index.shape[1]
    etot = e + n

    npad = ((n + 1 + 255) // 256) * 256
    unit = NW * CG
    epad = ((etot + unit - 1) // unit) * unit
    nchu = epad // unit
    c1 = ((nchu // 9) // 2) * 2 + 1
    cm = (nchu - 2 * c1) // 3
    cs = [c1, cm, cm, nchu - 2 * c1 - 2 * cm, c1]
    assert sum(cs) == nchu and all(c % 2 == 1 and c >= 3 for c in cs), cs

    loop = jnp.arange(n, dtype=jnp.int32)
    pad_e = epad - etot
    src_full = jnp.concatenate([
        edge_index[0].astype(jnp.int32), loop,
        jnp.zeros((pad_e,), jnp.int32)])
    dst_full = jnp.concatenate([
        edge_index[1].astype(jnp.int32), loop,
        n + (jnp.arange(pad_e, dtype=jnp.int32) % (npad - n))])

    xp = jnp.pad(x, ((0, npad - n), (0, 0)))
    posp = jnp.pad(pos, ((0, npad - n), (0, 16 - pos.shape[1])))
    WiT = Wi.T
    WlinT = Wlin.T
    WsaT = (Wa1 @ Wsrc).T
    WdaT = (Wa1 @ Wdst).T
    Wp1T16 = jnp.pad(Wp1.T, ((0, 16 - Wp1.shape[1]), (0, 0)))
    Wp1a = jnp.pad(Wp1.T, ((0, 8 - Wp1.shape[1]), (0, 0)))
    Wp1pad = jnp.zeros((128, 64), F32).at[64:64 + Wp1.shape[1]].set(Wp1.T)
    M64 = jnp.full((64, 64), 1.0 / 64.0, F32)
    Wp2T = Wp2.T
    Wa1T = Wa1.T
    Wa2T = Wa2.T
    WoT = Wo.T
    r1 = lambda v: v.reshape(1, -1)

    srctab, dsttab = _node_proj(xp, posp, WiT, r1(bi), WlinT, WsaT, WdaT,
                                Wp1T16)
    nums, dens = [], []
    off = 0
    for ck in cs:
        sl = slice(off * unit, (off + ck) * unit)
        off += ck
        gsrc, gdst = _sc_gather(src_full[sl], dst_full[sl], srctab, dsttab)
        msg, ex = _edge_mlp(gsrc, gdst, Wp1pad, Wp1a, r1(bp1), r1(gp),
                            r1(betap), Wp2T, r1(bp2), Wa1T, r1(ba1), r1(ga),
                            r1(betaa), Wa2T, r1(ba2), M64)
        num, den = _sc_scatter(msg, ex, dst_full[sl], npad)
        nums.append(num)
        dens.append(den)
    out = _final(nums, dens, WoT, r1(bo))
    return out[:n]


# speedup vs baseline: 11.0709x; 1.0025x over previous
import functools

import jax
import jax.numpy as jnp
from jax import lax
from jax.experimental import pallas as pl
from jax.experimental.pallas import tpu as pltpu
from jax.experimental.pallas import tpu_sc as plsc

F32 = jnp.float32

SC_CORES = 2
SC_SUBCORES = 16
NW = SC_CORES * SC_SUBCORES
CG = 128
CS = 128


def _ln_mm(v, g, b, m):
    mu = _dot(v, m)
    d = v - mu
    var = _dot(d * d, m)
    return d * lax.rsqrt(var + 1e-5) * g + b


def _dot(a, b):
    return jnp.dot(a, b, preferred_element_type=F32)


def _pack2(a, b):
    ua = (lax.bitcast_convert_type(a, jnp.uint32) + jnp.uint32(0x8000)) >> 16
    ub = (lax.bitcast_convert_type(b, jnp.uint32) + jnp.uint32(0x8000)) & jnp.uint32(0xFFFF0000)
    return lax.bitcast_convert_type(ua | ub, jnp.int32)


def _unpack2(u):
    uu = lax.bitcast_convert_type(u, jnp.uint32)
    a = lax.bitcast_convert_type(uu << 16, F32)
    b = lax.bitcast_convert_type(uu & jnp.uint32(0xFFFF0000), F32)
    return a, b


def _node_proj(xp, posp, WiT, bi2, WlinT, WsaT, WdaT, Wp1T16):
    npad, d = xp.shape
    blk = 1280
    grid = npad // blk

    def body(x_ref, pos_ref, wi, bi, wlin, wsa, wda, wp1, src_ref, dst_ref):
        x1 = jnp.maximum(_dot(x_ref[...], wi[...]) + bi[...], 0.0)
        h = _dot(x1, wlin[...])
        a1s = _dot(x1, wsa[...])
        psv = pos_ref[...]
        src_ref[:, 0:64] = _pack2(h[:, 0:64], h[:, 64:128])
        src_ref[:, 64:96] = _pack2(a1s[:, 0:32], a1s[:, 32:64])
        src_ref[:, 96:104] = _pack2(psv[:, 0:8], psv[:, 8:16])
        src_ref[:, 104:128] = jnp.zeros((blk, 24), jnp.int32)
        dst_ref[:, 0:64] = _dot(x1, wda[...])
        dst_ref[:, 64:80] = psv
        dst_ref[:, 80:128] = jnp.zeros((blk, 48), F32)

    full = lambda s: pl.BlockSpec(s, lambda i: (0, 0))
    return pl.pallas_call(
        body,
        grid=(grid,),
        in_specs=[
            pl.BlockSpec((blk, d), lambda i: (i, 0)),
            pl.BlockSpec((blk, 16), lambda i: (i, 0)),
            full((d, d)), full((1, d)), full((d, d)), full((d, 64)), full((d, 64)),
            full((16, 64)),
        ],
        out_specs=[
            pl.BlockSpec((blk, 128), lambda i: (i, 0)),
            pl.BlockSpec((blk, 128), lambda i: (i, 0)),
        ],
        out_shape=[
            jax.ShapeDtypeStruct((npad, 128), jnp.int32),
            jax.ShapeDtypeStruct((npad, 128), F32),
        ],
    )(xp, posp, WiT, bi2, WlinT, WsaT, WdaT, Wp1T16)


def _edge_mlp(gsrc, gdst, Wp1pad, Wp1a, bp1, gp, betap, Wp2T, bp2, Wa1T,
              ba1, ga, betaa, Wa2T, ba2, M64):
    epad = gsrc.shape[0]
    blk = 1024
    grid = epad // blk

    def body(gs_ref, gd_ref, wp1p, wp1a, bp1r, gpr, bpr, wp2, bp2r, wa1,
             ba1r, gar, bar, wa2, ba2r, m64, msg_ref, ex_ref):
        h0, h1 = _unpack2(gs_ref[:, 0:64])
        s0, s1 = _unpack2(gs_ref[:, 64:96])
        p0, _ = _unpack2(gs_ref[:, 96:104])
        gd = gd_ref[...]
        a1d = gd_ref[:, 0:64]
        t = _dot(gd, wp1p[...]) - _dot(p0, wp1a[...]) + bp1r[...]
        t = jnp.maximum(_ln_mm(t, gpr[...], bpr[...], m64[...]), 0.0)
        delta = _dot(t, wp2[...]) + bp2r[...]
        a1s = jnp.concatenate([s0, s1], axis=-1)
        a = a1d - a1s + _dot(delta, wa1[...]) + ba1r[...]
        a = jnp.maximum(_ln_mm(a, gar[...], bar[...], m64[...]), 0.0)
        alpha = _dot(a, wa2[...]) + ba2r[...]
        ex = jnp.exp(alpha)
        ex_ref[...] = ex
        msg_ref[:, 0:64] = ex[:, 0:64] * (h0 + delta[:, 0:64])
        msg_ref[:, 64:128] = ex[:, 64:128] * (h1 + delta[:, 64:128])

    full = lambda s: pl.BlockSpec(s, lambda i: (0, 0))
    return pl.pallas_call(
        body,
        grid=(grid,),
        in_specs=[
            pl.BlockSpec((blk, 128), lambda i: (i, 0)),
            pl.BlockSpec((blk, 128), lambda i: (i, 0)),
            full((128, 64)), full((8, 64)),
            full((1, 64)), full((1, 64)), full((1, 64)),
            full((64, 128)), full((1, 128)),
            full((128, 64)), full((1, 64)), full((1, 64)), full((1, 64)),
            full((64, 128)), full((1, 128)), full((64, 64)),
        ],
        out_specs=[
            pl.BlockSpec((blk, 128), lambda i: (i, 0)),
            pl.BlockSpec((blk, 128), lambda i: (i, 0)),
        ],
        out_shape=[
            jax.ShapeDtypeStruct((epad, 128), F32),
            jax.ShapeDtypeStruct((epad, 128), F32),
        ],
    )(gsrc, gdst, Wp1pad, Wp1a, bp1, gp, betap, Wp2T, bp2, Wa1T, ba1, ga,
      betaa, Wa2T, ba2, M64)


def _final(nums, dens, WoT, bo2):
    npad, d = nums[0].shape
    blk = 1280
    grid = npad // blk

    k = len(nums)

    def body(*refs):
        nrefs = refs[:k]
        drefs = refs[k:2 * k]
        wo, bo, out_ref = refs[2 * k], refs[2 * k + 1], refs[2 * k + 2]
        num = sum(r[...] for r in nrefs[1:]) + nrefs[0][...]
        den = sum(r[...] for r in drefs[1:]) + drefs[0][...]
        r = num / (den + 1e-16)
        out_ref[...] = jnp.maximum(_dot(r, wo[...]) + bo[...], 0.0)

    full = lambda s: pl.BlockSpec(s, lambda i: (0, 0))
    row = pl.BlockSpec((blk, d), lambda i: (i, 0))
    return pl.pallas_call(
        body,
        grid=(grid,),
        in_specs=[row] * (2 * k) + [full((d, d)), full((1, d))],
        out_specs=row,
        out_shape=jax.ShapeDtypeStruct((npad, d), F32),
    )(*nums, *dens, WoT, bo2)


def _sc_gather(src_ids, dst_ids, src_tab, dst_tab):
    epad = src_ids.shape[0]
    wd = dst_tab.shape[1]
    bpw = epad // NW
    nch = bpw // CG
    mesh = plsc.VectorSubcoreMesh(core_axis_name="c", subcore_axis_name="s")

    assert nch % 2 == 1 and nch >= 3, nch
    npairs = (nch - 1) // 2
    assert epad % 1024 == 0

    @functools.partial(
        pl.kernel,
        out_type=(
            jax.ShapeDtypeStruct((epad, 128), jnp.int32),
            jax.ShapeDtypeStruct((epad, wd), F32),
        ),
        mesh=mesh,
        scratch_types=[
            [pltpu.VMEM((CG,), jnp.int32)] * 2,
            [pltpu.VMEM((CG,), jnp.int32)] * 2,
            [pltpu.VMEM((CG, 128), jnp.int32)] * 2,
            [pltpu.VMEM((CG, wd), F32)] * 2,
            [pltpu.SemaphoreType.DMA] * 2,
            [pltpu.SemaphoreType.DMA] * 2,
        ],
    )
    def k(sid_h, did_h, stab_h, dtab_h, gsrc_h, gdst_h, idx_s, idx_d, buf_s,
          buf_d, sem_g, sem_w):
        wid = lax.axis_index("s") * SC_CORES + lax.axis_index("c")
        base = wid * bpw

        def g_start(off, t):
            pltpu.sync_copy(sid_h.at[pl.ds(off, CG)], idx_s[t])
            pltpu.sync_copy(did_h.at[pl.ds(off, CG)], idx_d[t])
            pltpu.async_copy(stab_h.at[idx_s[t]], buf_s[t], sem_g[t])
            pltpu.async_copy(dtab_h.at[idx_d[t]], buf_d[t], sem_g[t])

        def g_wait(t):
            pltpu.make_async_copy(stab_h.at[idx_s[t]], buf_s[t], sem_g[t]).wait()
            pltpu.make_async_copy(dtab_h.at[idx_d[t]], buf_d[t], sem_g[t]).wait()

        def w_start(off, t):
            pltpu.async_copy(buf_s[t], gsrc_h.at[pl.ds(off, CG)], sem_w[t])
            pltpu.async_copy(buf_d[t], gdst_h.at[pl.ds(off, CG)], sem_w[t])

        def w_wait(off, t):
            pltpu.make_async_copy(buf_s[t], gsrc_h.at[pl.ds(off, CG)], sem_w[t]).wait()
            pltpu.make_async_copy(buf_d[t], gdst_h.at[pl.ds(off, CG)], sem_w[t]).wait()

        g_start(base, 0)

        def body(p, carry):
            offa = base + (2 * p) * CG
            offb = offa + CG
            offn = offb + CG
            g_wait(0)
            pl.when(p > 0)(lambda: w_wait(offa - CG, 1))
            g_start(offb, 1)
            w_start(offa, 0)
            g_wait(1)
            w_start(offb, 1)
            w_wait(offa, 0)
            g_start(offn, 0)
            return carry

        lax.fori_loop(0, npairs, body, 0)
        last = base + (nch - 1) * CG
        g_wait(0)
        w_wait(last - CG, 1)
        w_start(last, 0)
        w_wait(last, 0)

    return k(src_ids, dst_ids, src_tab, dst_tab)


def _sc_scatter(msg, ex, dst_ids, npad):
    epad, d = msg.shape
    per_tile = epad // SC_SUBCORES
    nch = per_tile // CS
    rows_per_tile = npad // SC_SUBCORES
    mesh = plsc.VectorSubcoreMesh(core_axis_name="c", subcore_axis_name="s")

    @functools.partial(
        pl.kernel,
        out_type=(
            jax.ShapeDtypeStruct((npad, d), F32),
            jax.ShapeDtypeStruct((npad, d), F32),
        ),
        mesh=mesh,
        scratch_types=[
            [pltpu.VMEM((CS, d), F32)] * 2,
            [pltpu.VMEM((CS,), jnp.int32)] * 2,
            pltpu.VMEM((64, d), F32),
            pltpu.VMEM_SHARED((npad, d), F32),
            [pltpu.SemaphoreType.DMA] * 2,
            [pltpu.SemaphoreType.DMA] * 2,
        ],
    )
    def k(msg_h, ex_h, did_h, num_h, den_h, rowbuf, idxb, zbuf, acc, sem_l,
          sem_a):
        c = lax.axis_index("c")
        s = lax.axis_index("s")

        for j in range(64):
            for t in range(d // 16):
                zbuf[j, pl.ds(t * 16, 16)] = jnp.zeros((16,), F32)

        def zero_start(i, carry):
            pltpu.async_copy(
                zbuf, acc.at[pl.ds(s * rows_per_tile + i * 64, 64)], sem_l[0])
            return carry

        def zero_wait(i, carry):
            pltpu.make_async_copy(
                zbuf, acc.at[pl.ds(s * rows_per_tile + i * 64, 64)],
                sem_l[0]).wait()
            return carry

        lax.fori_loop(0, rows_per_tile // 64, zero_start, 0)
        lax.fori_loop(0, rows_per_tile // 64, zero_wait, 0)
        plsc.subcore_barrier()

        def process(tbl_h):
            base = s * per_tile

            def l_start(off, t):
                pltpu.sync_copy(did_h.at[pl.ds(off, CS)], idxb[t])
                pltpu.async_copy(tbl_h.at[pl.ds(off, CS)], rowbuf[t], sem_l[t])

            def l_wait(off, t):
                pltpu.make_async_copy(
                    tbl_h.at[pl.ds(off, CS)], rowbuf[t], sem_l[t]).wait()

            def a_start(t):
                pltpu.async_copy(rowbuf[t], acc.at[idxb[t]], sem_a[t], add=True)

            def a_wait(t):
                pltpu.make_async_copy(
                    rowbuf[t], acc.at[idxb[t]], sem_a[t]).wait()

            l_start(base, 0)

            def body(p, carry):
                offa = base + (2 * p) * CS
                offb = offa + CS
                offn = offb + CS
                pl.when(p > 0)(lambda: a_wait(1))
                l_start(offb, 1)
                l_wait(offa, 0)
                a_start(0)
                l_wait(offb, 1)
                a_start(1)
                a_wait(0)
                pl.when(p + 1 < nch // 2)(lambda: l_start(offn, 0))
                return carry

            lax.fori_loop(0, nch // 2, body, 0)
            a_wait(1)

        pl.when(c == 0)(lambda: process(msg_h))
        pl.when(c == 1)(lambda: process(ex_h))
        plsc.subcore_barrier()

        def dump(out_h):
            pltpu.sync_copy(
                acc.at[pl.ds(s * rows_per_tile, rows_per_tile)],
                out_h.at[pl.ds(s * rows_per_tile, rows_per_tile)])

        pl.when(c == 0)(lambda: dump(num_h))
        pl.when(c == 1)(lambda: dump(den_h))

    return k(msg, ex, dst_ids)


def kernel(x, pos, edge_index, Wi, bi, Wo, bo, Wlin, Wsrc, Wdst, Wp1, bp1,
           gp, betap, Wp2, bp2, Wa1, ba1, ga, betaa, Wa2, ba2):
    n, dm = x.shape
    e = edge_index.shape[1]
    etot = e + n

    npad = ((n + 1 + 255) // 256) * 256
    unit = NW * CG
    epad = ((etot + unit - 1) // unit) * unit
    nchu = epad // unit
    c1 = ((nchu // 9) // 2) * 2 + 1
    cm = (nchu - 2 * c1) // 3
    cs = [c1, cm, cm, nchu - 2 * c1 - 2 * cm, c1]
    assert sum(cs) == nchu and all(c % 2 == 1 and c >= 3 for c in cs), cs

    loop = jnp.arange(n, dtype=jnp.int32)
    pad_e = epad - etot
    src_full = jnp.concatenate([
        edge_index[0].astype(jnp.int32), loop,
        jnp.zeros((pad_e,), jnp.int32)])
    dst_full = jnp.concatenate([
        edge_index[1].astype(jnp.int32), loop,
        n + (jnp.arange(pad_e, dtype=jnp.int32) % (npad - n))])

    xp = jnp.pad(x, ((0, npad - n), (0, 0)))
    posp = jnp.pad(pos, ((0, npad - n), (0, 16 - pos.shape[1])))
    WiT = Wi.T
    WlinT = Wlin.T
    WsaT = (Wa1 @ Wsrc).T
    WdaT = (Wa1 @ Wdst).T
    Wp1T16 = jnp.pad(Wp1.T, ((0, 16 - Wp1.shape[1]), (0, 0)))
    Wp1a = jnp.pad(Wp1.T, ((0, 8 - Wp1.shape[1]), (0, 0)))
    Wp1pad = jnp.zeros((128, 64), F32).at[64:64 + Wp1.shape[1]].set(Wp1.T)
    M64 = jnp.full((64, 64), 1.0 / 64.0, F32)
    Wp2T = Wp2.T
    Wa1T = Wa1.T
    Wa2T = Wa2.T
    WoT = Wo.T
    r1 = lambda v: v.reshape(1, -1)

    srctab, dsttab = _node_proj(xp, posp, WiT, r1(bi), WlinT, WsaT, WdaT,
                                Wp1T16)
    nums, dens = [], []
    off = 0
    for ck in cs:
        sl = slice(off * unit, (off + ck) * unit)
        off += ck
        gsrc, gdst = _sc_gather(src_full[sl], dst_full[sl], srctab, dsttab)
        msg, ex = _edge_mlp(gsrc, gdst, Wp1pad, Wp1a, r1(bp1), r1(gp),
                            r1(betap), Wp2T, r1(bp2), Wa1T, r1(ba1), r1(ga),
                            r1(betaa), Wa2T, r1(ba2), M64)
        num, den = _sc_scatter(msg, ex, dst_full[sl], npad)
        nums.append(num)
        dens.append(den)
    out = _final(nums, dens, WoT, r1(bo))
    return out[:n]
